# Initial kernel scaffold; baseline (speedup 1.0000x reference)
import jax
import jax.numpy as jnp
from jax.experimental import pallas as pl


def kernel(x, pos, edge_index, batch, W_down0, b_down0, W_down1, b_down1, W_up0, b_up0, W_up1, b_up1, W_lin, b_lin):
    raise NotImplementedError("write your pallas kernel here")


# speedup vs baseline: 9.3499x; 9.3499x over previous
import functools

import jax
import jax.numpy as jnp
from jax import lax
from jax.experimental import pallas as pl
from jax.experimental.pallas import tpu as pltpu
from jax.experimental.pallas import tpu_sc as plsc

N = 10000
E = 320000
HID = 128
NPAD = 10240
M1, M1PAD = 7500, 7680
M2, M2PAD = 5625, 6144
K1PAD = 7504
K2PAD = 5632
DUMP1 = 7600
DUMP0 = 10100
PADPOS = 1.0e15
S8 = 0.35355339059327373

@functools.lru_cache(maxsize=None)
def _mesh():
    return plsc.VectorSubcoreMesh(
        core_axis_name="c", subcore_axis_name="s", num_cores=2,
        num_subcores=16)


def _zero_vmem2d(ref, rows, width):
    per = width // 16

    def body(i, _):
        r = i // per
        l = i % per
        ref[r, pl.ds(l * 16, 16)] = jnp.zeros((16,), ref.dtype)
        return 0

    lax.fori_loop(0, rows * per, body, 0)


def _zero_vmem1d(ref, n):
    def body(i, _):
        ref[pl.ds(i * 16, 16)] = jnp.zeros((16,), ref.dtype)
        return 0

    lax.fori_loop(0, n // 16, body, 0)


def make_hist(R, EC, rpc, CH, NB):
    PT = EC // 16
    NCH = PT // CH
    SL = NB // 16

    @functools.partial(
        pl.kernel,
        mesh=_mesh(),
        compiler_params=pltpu.CompilerParams(needs_layout_passes=False),
        out_type=jax.ShapeDtypeStruct((2 * NB,), jnp.float32),
        scratch_types=[
            pltpu.VMEM((NB,), jnp.float32),
            pltpu.VMEM((CH,), jnp.int32),
            pltpu.VMEM((SL,), jnp.float32),
            pltpu.VMEM((SL,), jnp.float32),
            pltpu.VMEM_SHARED((16, NB), jnp.float32),
        ],
    )
    def k(idx2, out, histv, idxv, accv, tmpv, stage):
        c = lax.axis_index("c")
        s = lax.axis_index("s")
        _zero_vmem1d(histv, NB)
        ones = jnp.ones((16,), jnp.float32)
        for r in range(rpc):
            row = c * rpc + r
            for kk in range(NCH):
                base = row * EC + s * PT + kk * CH
                pltpu.sync_copy(idx2.at[pl.ds(base, CH)], idxv)

                def sbody(j, _):
                    iv = idxv[pl.ds(j * 16, 16)]
                    plsc.addupdate_scatter(histv, [iv], ones)
                    return 0

                lax.fori_loop(0, CH // 16, sbody, 0)
        pltpu.sync_copy(histv, stage.at[s])
        plsc.subcore_barrier()
        off = s * SL
        pltpu.sync_copy(stage.at[0, pl.ds(off, SL)], accv)
        for t in range(1, 16):
            pltpu.sync_copy(stage.at[t, pl.ds(off, SL)], tmpv)

            def abody(q, _):
                sl = pl.ds(q * 16, 16)
                accv[sl] = accv[sl] + tmpv[sl]
                return 0

            lax.fori_loop(0, SL // 16, abody, 0)
        pltpu.sync_copy(accv, out.at[pl.ds(c * NB + off, SL)])

    return k


def make_agg_scatter():
    EPT = E // 32
    NFULL = EPT // 128
    TAIL = EPT - NFULL * 128

    @functools.partial(
        pl.kernel,
        mesh=_mesh(),
        compiler_params=pltpu.CompilerParams(needs_layout_passes=False),
        out_type=jax.ShapeDtypeStruct((2, NPAD, HID), jnp.float32),
        scratch_types=[
            pltpu.VMEM((2, 128), jnp.int32),
            pltpu.VMEM((2, 16), jnp.int32),
            pltpu.VMEM((128, HID), jnp.float32),
            pltpu.VMEM((128, HID), jnp.float32),
            pltpu.VMEM_SHARED((NPAD, HID), jnp.float32),
            pltpu.SemaphoreType.DMA,
        ],
    )
    def k(y, ei, out, idxs, idxt, rowsv, zbuf, acc, sem):
        c = lax.axis_index("c")
        s = lax.axis_index("s")
        wid = c * 16 + s
        _zero_vmem2d(zbuf, 128, HID)
        for b in range(5):
            pltpu.sync_copy(zbuf, acc.at[pl.ds(s * 640 + b * 128, 128)])
        plsc.subcore_barrier()
        for kk in range(NFULL):
            base = wid * EPT + kk * 128
            pltpu.sync_copy(ei.at[pl.ds(base, 128)], idxs.at[0])
            pltpu.sync_copy(ei.at[pl.ds(E + base, 128)], idxs.at[1])
            pltpu.async_copy(y.at[idxs.at[0]], rowsv, sem).wait()
            pltpu.sync_copy(rowsv, acc.at[idxs.at[1]], add=True)
        base = wid * EPT + NFULL * 128
        pltpu.sync_copy(ei.at[pl.ds(base, TAIL)], idxt.at[0])
        pltpu.sync_copy(ei.at[pl.ds(E + base, TAIL)], idxt.at[1])
        pltpu.async_copy(y.at[idxt.at[0]], rowsv.at[pl.ds(0, TAIL)], sem).wait()
        pltpu.sync_copy(rowsv.at[pl.ds(0, TAIL)], acc.at[idxt.at[1]], add=True)
        plsc.subcore_barrier()
        for b in range(5):
            sl = pl.ds(s * 640 + b * 128, 128)
            pltpu.sync_copy(acc.at[sl], out.at[c, sl])

    return k


def make_agg_knn():
    CNT = M1PAD // 32
    CH = 120

    @functools.partial(
        pl.kernel,
        mesh=_mesh(),
        compiler_params=pltpu.CompilerParams(needs_layout_passes=False),
        out_type=jax.ShapeDtypeStruct((M1PAD, HID), jnp.float32),
        scratch_types=[
            pltpu.VMEM((6, CH), jnp.int32),
            pltpu.VMEM((CH, HID), jnp.float32),
            pltpu.VMEM((CH, HID), jnp.float32),
            pltpu.SemaphoreType.DMA,
        ],
    )
    def k(yy, nbrT, out, idxv, rowsv, accv, sem):
        c = lax.axis_index("c")
        s = lax.axis_index("s")
        wid = c * 16 + s
        for t in range(CNT // CH):
            base = wid * CNT + t * CH
            for j in range(6):
                pltpu.sync_copy(nbrT.at[pl.ds(j * M1PAD + base, CH)],
                                idxv.at[j])
            pltpu.async_copy(yy.at[idxv.at[0]], accv, sem).wait()
            for j in range(1, 6):
                pltpu.async_copy(yy.at[idxv.at[j]], rowsv, sem).wait()

                def abody(i, _):
                    r = i // 8
                    l = i % 8
                    sl = pl.ds(l * 16, 16)
                    accv[r, sl] = accv[r, sl] + rowsv[r, sl]
                    return 0

                lax.fori_loop(0, CH * 8, abody, 0)
            pltpu.sync_copy(accv, out.at[pl.ds(base, CH)])

    return k


def make_topk(nvalid, K, bias, KOUT):
    NQ = NPAD // 16

    @functools.partial(
        pl.kernel,
        mesh=_mesh(),
        compiler_params=pltpu.CompilerParams(needs_layout_passes=False),
        out_type=jax.ShapeDtypeStruct((KOUT,), jnp.int32),
        scratch_types=[
            pltpu.VMEM((2 * NPAD,), jnp.float32),
            pltpu.VMEM((NPAD,), jnp.float32),
            pltpu.VMEM((KOUT + 16,), jnp.int32),
        ],
    )
    def k(histpair, out, hpv, scorev, outv):
        c = lax.axis_index("c")
        s = lax.axis_index("s")

        @pl.when(jnp.logical_and(c == 0, s == 0))
        def _():
            pltpu.sync_copy(histpair, hpv)
            iota16 = lax.broadcasted_iota(jnp.int32, (16,), 0)

            def build(q, _):
                sl = pl.ds(q * 16, 16)
                gidx = q * 16 + iota16
                sv = (hpv[pl.ds(q * 16, 16)]
                      + hpv[pl.ds(NPAD + q * 16, 16)] + jnp.float32(bias))
                scorev[sl] = jnp.where(gidx < nvalid, sv, -1.0)
                return 0

            lax.fori_loop(0, NQ, build, 0)

            def count_ge(thr):
                def cbody(q, acc):
                    sv = scorev[pl.ds(q * 16, 16)]
                    return acc + jnp.where(sv >= thr, 1, 0)

                accv = lax.fori_loop(0, NQ, cbody, jnp.zeros((16,), jnp.int32))
                return jnp.sum(accv)

            def bbody(_, lohi):
                lo, hi = lohi
                mid = (lo + hi) // 2
                cnt = count_ge(mid.astype(jnp.float32))
                big = cnt >= K
                return (jnp.where(big, mid, lo), jnp.where(big, hi, mid))

            lo, hi = lax.fori_loop(
                0, 21, bbody, (jnp.int32(0), jnp.int32(1 << 20))
            )
            Tf = lo.astype(jnp.float32)
            n_gt = count_ge(Tf + 0.5)
            need_eq = K - n_gt

            def comp(q, carry):
                pos, eqseen = carry
                sl = pl.ds(q * 16, 16)
                sv = scorev[sl]
                gidx = q * 16 + iota16
                m_gt = sv >= Tf + 0.5
                m_eq = jnp.logical_and(sv >= Tf - 0.5, sv <= Tf + 0.5)
                eqc = plsc.cumsum(m_eq.astype(jnp.int32))
                keep = jnp.logical_or(
                    m_gt, jnp.logical_and(m_eq, (eqseen + eqc) <= need_eq)
                )
                plsc.store_compressed(outv.at[pl.ds(pos, 16)], gidx, mask=keep)
                pos = pos + jnp.sum(keep.astype(jnp.int32))
                eqseen = eqseen + jnp.sum(m_eq.astype(jnp.int32))
                return (pos, eqseen)

            lax.fori_loop(0, NQ, comp, (jnp.int32(0), jnp.int32(0)))
            pltpu.sync_copy(outv.at[pl.ds(0, KOUT)], out)

    return k


def make_gather_rows(B, CH):
    CNT = B // 32

    @functools.partial(
        pl.kernel,
        mesh=_mesh(),
        compiler_params=pltpu.CompilerParams(needs_layout_passes=False),
        out_type=jax.ShapeDtypeStruct((B, HID), jnp.float32),
        scratch_types=[
            pltpu.VMEM((CH,), jnp.int32),
            pltpu.VMEM((CH, HID), jnp.float32),
            pltpu.SemaphoreType.DMA,
        ],
    )
    def k(table, idx, out, idxv, rowsv, sem):
        c = lax.axis_index("c")
        s = lax.axis_index("s")
        wid = c * 16 + s
        for t in range(CNT // CH):
            base = wid * CNT + t * CH
            pltpu.sync_copy(idx.at[pl.ds(base, CH)], idxv)
            pltpu.async_copy(table.at[idxv], rowsv, sem).wait()
            pltpu.sync_copy(rowsv, out.at[pl.ds(base, CH)])

    return k


def make_gather_cols():
    CNT = M1PAD // 32

    @functools.partial(
        pl.kernel,
        mesh=_mesh(),
        compiler_params=pltpu.CompilerParams(needs_layout_passes=False),
        out_type=jax.ShapeDtypeStruct((4 * M1PAD,), jnp.float32),
        scratch_types=[
            pltpu.VMEM((NPAD,), jnp.float32),
            pltpu.VMEM((CNT,), jnp.int32),
            pltpu.VMEM((CNT,), jnp.float32),
        ],
    )
    def k(colsT, idx, out, colv, idxv, outvv):
        c = lax.axis_index("c")
        s = lax.axis_index("s")
        wid = c * 16 + s
        base = wid * CNT
        iota16 = lax.broadcasted_iota(jnp.int32, (16,), 0)
        pltpu.sync_copy(idx.at[pl.ds(base, CNT)], idxv)
        for cc in range(4):
            pltpu.sync_copy(colsT.at[pl.ds(cc * NPAD, NPAD)], colv)
            fill = jnp.float32(PADPOS if cc < 3 else 0.0)

            def gbody(q, _):
                sl = pl.ds(q * 16, 16)
                iv = idxv[sl]
                vals = plsc.load_gather(colv, [iv])
                gidx = base + q * 16 + iota16
                outvv[sl] = jnp.where(gidx < M1, vals, fill)
                return 0

            lax.fori_loop(0, CNT // 16, gbody, 0)
            pltpu.sync_copy(outvv, out.at[pl.ds(cc * M1PAD + base, CNT)])

    return k


def make_scatter_rows(B, OUTR, CH, ZB):
    CNT = B // 16
    NCH = CNT // CH
    ZR = OUTR // 16

    @functools.partial(
        pl.kernel,
        mesh=_mesh(),
        compiler_params=pltpu.CompilerParams(needs_layout_passes=False),
        out_type=jax.ShapeDtypeStruct((OUTR, HID), jnp.float32),
        scratch_types=[
            pltpu.VMEM((NCH, CH), jnp.int32),
            pltpu.VMEM((CH, HID), jnp.float32),
            pltpu.VMEM((ZB, HID), jnp.float32),
        ],
    )
    def k(z, idx, out, idxv, rowsv, zbuf):
        c = lax.axis_index("c")
        s = lax.axis_index("s")

        @pl.when(c == 0)
        def _():
            _zero_vmem2d(zbuf, ZB, HID)
            for b in range(ZR // ZB):
                pltpu.sync_copy(zbuf, out.at[pl.ds(s * ZR + b * ZB, ZB)])
            plsc.subcore_barrier()
            for kk in range(NCH):
                base = s * CNT + kk * CH
                pltpu.sync_copy(idx.at[pl.ds(base, CH)], idxv.at[kk])
                pltpu.sync_copy(z.at[pl.ds(base, CH)], rowsv)
                pltpu.sync_copy(rowsv, out.at[idxv.at[kk]])

    return k


def make_mm(nrows, with_hist=False, with_scale=False, with_bias=False,
            relu=False):
    grid = nrows // 128

    def body(*refs):
        i = 0
        a_ref = refs[i]; i += 1
        w_ref = refs[i]; i += 1
        h_ref = None
        sc_ref = None
        b_ref = None
        if with_hist:
            h_ref = refs[i]; i += 1
        if with_scale:
            sc_ref = refs[i]; i += 1
        if with_bias:
            b_ref = refs[i]; i += 1
        out_ref = refs[i]; i += 1
        acc = jnp.dot(a_ref[...], w_ref[...],
                      preferred_element_type=jnp.float32)
        if with_hist:
            dinv = lax.rsqrt(h_ref[...] + 2.0)
            refs[i][...] = dinv
            acc = acc * dinv
        if with_scale:
            acc = acc * sc_ref[...]
        if with_bias:
            acc = acc + b_ref[...]
        if relu:
            acc = jnp.maximum(acc, 0.0)
        out_ref[...] = acc

    in_specs = [
        pl.BlockSpec((128, HID), lambda i: (i, 0)),
        pl.BlockSpec((HID, HID), lambda i: (0, 0)),
    ]
    if with_hist:
        in_specs.append(pl.BlockSpec((128, 1), lambda i: (i, 0)))
    if with_scale:
        in_specs.append(pl.BlockSpec((128, 1), lambda i: (i, 0)))
    if with_bias:
        in_specs.append(pl.BlockSpec((1, HID), lambda i: (0, 0)))
    out_shape = [jax.ShapeDtypeStruct((nrows, HID), jnp.float32)]
    out_specs = [pl.BlockSpec((128, HID), lambda i: (i, 0))]
    if with_hist:
        out_shape.append(jax.ShapeDtypeStruct((nrows, 1), jnp.float32))
        out_specs.append(pl.BlockSpec((128, 1), lambda i: (i, 0)))
    return pl.pallas_call(
        body,
        grid=(grid,),
        in_specs=in_specs,
        out_specs=out_specs if len(out_specs) > 1 else out_specs[0],
        out_shape=out_shape if len(out_shape) > 1 else out_shape[0],
    )


def make_post2(nrows):
    def body(sa, sb, y, dinv, b, out):
        d = dinv[...]
        out[...] = jnp.maximum(
            d * (sa[...] + sb[...]) + 2.0 * d * y[...] + b[...], 0.0
        )

    return pl.pallas_call(
        body,
        grid=(nrows // 128,),
        in_specs=[
            pl.BlockSpec((128, HID), lambda i: (i, 0)),
            pl.BlockSpec((128, HID), lambda i: (i, 0)),
            pl.BlockSpec((128, HID), lambda i: (i, 0)),
            pl.BlockSpec((128, 1), lambda i: (i, 0)),
            pl.BlockSpec((1, HID), lambda i: (0, 0)),
        ],
        out_specs=pl.BlockSpec((128, HID), lambda i: (i, 0)),
        out_shape=jax.ShapeDtypeStruct((nrows, HID), jnp.float32),
    )


def make_post1(nrows):
    def body(sref, y, b, out):
        out[...] = jnp.maximum(
            S8 * sref[...] + (2.0 * S8) * y[...] + b[...], 0.0
        )

    return pl.pallas_call(
        body,
        grid=(nrows // 128,),
        in_specs=[
            pl.BlockSpec((128, HID), lambda i: (i, 0)),
            pl.BlockSpec((128, HID), lambda i: (i, 0)),
            pl.BlockSpec((1, HID), lambda i: (0, 0)),
        ],
        out_specs=pl.BlockSpec((128, HID), lambda i: (i, 0)),
        out_shape=jax.ShapeDtypeStruct((nrows, HID), jnp.float32),
    )


def make_knn():
    BIGF = 3.0e38
    BIGI = 1 << 30

    def body(posR, posT, out):
        i = pl.program_id(0)
        pr = posR[...]
        a0 = pr[:, 0:1]
        a1 = pr[:, 1:2]
        a2 = pr[:, 2:3]
        c0 = posT[0:1, :]
        c1 = posT[1:2, :]
        c2 = posT[2:3, :]
        sq_r = a0 * a0 + a1 * a1 + a2 * a2
        sq_c = c0 * c0 + c1 * c1 + c2 * c2
        dots = lax.dot_general(pr, posT[...],
                               (((1,), (0,)), ((), ())))
        d = (sq_r + sq_c) - 2.0 * dots
        col = lax.broadcasted_iota(jnp.int32, (128, M1PAD), 1)
        row = i * 128 + lax.broadcasted_iota(jnp.int32, (128, M1PAD), 0)
        d = jnp.where(col == row, BIGF, d)
        for j in range(6):
            mn = jnp.min(d, axis=1, keepdims=True)
            amin = jnp.min(jnp.where(d == mn, col, BIGI), axis=1)
            out[j, :] = amin
            d = jnp.where(col == amin[:, None], BIGF, d)
        z = jnp.zeros((128,), jnp.int32)
        out[6, :] = z
        out[7, :] = z

    return pl.pallas_call(
        body,
        grid=(M1PAD // 128,),
        in_specs=[
            pl.BlockSpec((128, 3), lambda i: (i, 0)),
            pl.BlockSpec((3, M1PAD), lambda i: (0, 0)),
        ],
        out_specs=pl.BlockSpec((8, 128), lambda i: (0, i)),
        out_shape=jax.ShapeDtypeStruct((8, M1PAD), jnp.int32),
    )


@functools.lru_cache(maxsize=None)
def _ops():
    class O:
        pass
    o = O()
    o.hist_e0 = make_hist(2, E, 1, 2000, NPAD)
    o.hist_e1 = make_hist(8, M1PAD, 3, 480, NPAD)
    o.agg0 = make_agg_scatter()
    o.agg1 = make_agg_knn()
    o.topk1 = make_topk(N, M1, 0.0, K1PAD)
    o.topk2 = make_topk(M1, M2, 6.0, K2PAD)
    o.gather1 = make_gather_rows(M1PAD, 120)
    o.gather2 = make_gather_rows(M2PAD, 96)
    o.gather_cols = make_gather_cols()
    o.scatter_a = make_scatter_rows(K2PAD, M1PAD, 88, 120)
    o.scatter_b = make_scatter_rows(M1PAD, NPAD, 120, 128)
    o.mm_y = make_mm(NPAD, with_hist=True)
    o.mm_1 = make_mm(M1PAD)
    o.mm_2 = make_mm(M2PAD)
    o.mm_up1 = make_mm(M1PAD, with_scale=True)
    o.mm_out = make_mm(NPAD, with_bias=True)
    o.post2_0 = make_post2(NPAD)
    o.post1_1 = make_post1(M1PAD)
    o.knn = make_knn()
    return o


def kernel(x, pos, edge_index, batch, W_down0, b_down0, W_down1, b_down1,
           W_up0, b_up0, W_up1, b_up1, W_lin, b_lin):
    del batch
    o = _ops()
    ei = edge_index.astype(jnp.int32)
    xp = jnp.zeros((NPAD, HID), jnp.float32).at[:N].set(x)

    ei1d = ei.reshape(-1)
    histpair0 = o.hist_e0(ei1d)
    histd = histpair0[NPAD:].reshape(NPAD, 1)

    y, dinv0 = o.mm_y(xp, W_down0, histd)

    S = o.agg0(y, ei1d)
    h0 = o.post2_0(S[0], S[1], y, dinv0, b_down0.reshape(1, HID))

    perm1 = o.topk1(histpair0)
    perm1g = jnp.concatenate(
        [perm1[:M1], jnp.zeros((M1PAD - M1,), jnp.int32)])
    x1 = o.gather1(h0, perm1g)

    colsT = jnp.concatenate(
        [jnp.zeros((3, NPAD), jnp.float32).at[:, :N].set(pos.T),
         dinv0.reshape(1, NPAD)])
    cols1 = o.gather_cols(colsT.reshape(-1), perm1g).reshape(4, M1PAD)
    posT1 = cols1[:3]
    dinv0p = cols1[3].reshape(M1PAD, 1)

    nbrT = o.knn(posT1.T, posT1)

    nbr1d = nbrT.reshape(-1)
    y1 = o.mm_1(x1, W_down1 * S8)
    S1 = o.agg1(y1, nbr1d)
    h1 = o.post1_1(S1, y1, b_down1.reshape(1, HID))

    histpair1 = o.hist_e1(nbr1d)
    perm2 = o.topk2(histpair1)
    perm2g = jnp.concatenate(
        [perm2[:M2], jnp.zeros((M2PAD - M2,), jnp.int32)])
    x2 = o.gather2(h1, perm2g)

    z = o.mm_2(x2, W_up0 * S8)
    sidx2 = jnp.concatenate(
        [perm2[:M2], jnp.full((K2PAD - M2,), DUMP1, jnp.int32)])
    y_up = o.scatter_a(z[:K2PAD], sidx2)
    S2 = o.agg1(y_up, nbr1d)
    h_up0 = o.post1_1(S2, y_up, b_up0.reshape(1, HID))

    z2 = o.mm_up1(h_up0, W_up1, dinv0p)
    sidx1 = jnp.concatenate(
        [perm1[:M1], jnp.full((M1PAD - M1,), DUMP0, jnp.int32)])
    y2full = o.scatter_b(z2, sidx1)
    S3 = o.agg0(y2full, ei1d)
    h_up1 = o.post2_0(S3[0], S3[1], y2full, dinv0, b_up1.reshape(1, HID))

    out = o.mm_out(h_up1, W_lin, b_lin.reshape(1, HID))
    return out[:N]


# speedup vs baseline: 11.2644x; 1.2048x over previous
import functools

import jax
import jax.numpy as jnp
from jax import lax
from jax.experimental import pallas as pl
from jax.experimental.pallas import tpu as pltpu
from jax.experimental.pallas import tpu_sc as plsc

N = 10000
E = 320000
HID = 128
NPAD = 10240
M1, M1PAD = 7500, 7680
M2, M2PAD = 5625, 6144
K1PAD = 7504
K2PAD = 5632
DUMP1 = 7600
DUMP0 = 10100
PADPOS = 1.0e15
S8 = 0.35355339059327373

@functools.lru_cache(maxsize=None)
def _mesh():
    return plsc.VectorSubcoreMesh(
        core_axis_name="c", subcore_axis_name="s", num_cores=2,
        num_subcores=16)


def _zero_vmem2d(ref, rows, width):
    per = width // 16

    def body(i, _):
        r = i // per
        l = i % per
        ref[r, pl.ds(l * 16, 16)] = jnp.zeros((16,), ref.dtype)
        return 0

    lax.fori_loop(0, rows * per, body, 0)


def _zero_vmem1d(ref, n):
    def body(i, _):
        ref[pl.ds(i * 16, 16)] = jnp.zeros((16,), ref.dtype)
        return 0

    lax.fori_loop(0, n // 16, body, 0)


def make_hist(R, EC, rpc, CH, NB):
    PT = EC // 16
    NCH = PT // CH
    SL = NB // 16

    @functools.partial(
        pl.kernel,
        mesh=_mesh(),
        compiler_params=pltpu.CompilerParams(needs_layout_passes=False),
        out_type=jax.ShapeDtypeStruct((2 * NB,), jnp.float32),
        scratch_types=[
            pltpu.VMEM((NB,), jnp.float32),
            pltpu.VMEM((CH,), jnp.int32),
            pltpu.VMEM((SL,), jnp.float32),
            pltpu.VMEM((SL,), jnp.float32),
            pltpu.VMEM_SHARED((16, NB), jnp.float32),
        ],
    )
    def k(idx2, out, histv, idxv, accv, tmpv, stage):
        c = lax.axis_index("c")
        s = lax.axis_index("s")
        _zero_vmem1d(histv, NB)
        ones = jnp.ones((16,), jnp.float32)
        for r in range(rpc):
            row = c * rpc + r
            for kk in range(NCH):
                base = row * EC + s * PT + kk * CH
                pltpu.sync_copy(idx2.at[pl.ds(base, CH)], idxv)

                def sbody(j, _):
                    iv = idxv[pl.ds(j * 16, 16)]
                    plsc.addupdate_scatter(histv, [iv], ones)
                    return 0

                lax.fori_loop(0, CH // 16, sbody, 0)
        pltpu.sync_copy(histv, stage.at[s])
        plsc.subcore_barrier()
        off = s * SL
        pltpu.sync_copy(stage.at[0, pl.ds(off, SL)], accv)
        for t in range(1, 16):
            pltpu.sync_copy(stage.at[t, pl.ds(off, SL)], tmpv)

            def abody(q, _):
                sl = pl.ds(q * 16, 16)
                accv[sl] = accv[sl] + tmpv[sl]
                return 0

            lax.fori_loop(0, SL // 16, abody, 0)
        pltpu.sync_copy(accv, out.at[pl.ds(c * NB + off, SL)])

    return k


def make_agg_scatter():
    EPT = E // 32
    NFULL = EPT // 128
    TAIL = EPT - NFULL * 128

    @functools.partial(
        pl.kernel,
        mesh=_mesh(),
        compiler_params=pltpu.CompilerParams(needs_layout_passes=False),
        out_type=jax.ShapeDtypeStruct((2, NPAD, HID), jnp.float32),
        scratch_types=[
            pltpu.VMEM((4, 128), jnp.int32),
            pltpu.VMEM((2, 16), jnp.int32),
            pltpu.VMEM((128, HID), jnp.float32),
            pltpu.VMEM((128, HID), jnp.float32),
            pltpu.VMEM_SHARED((NPAD, HID), jnp.float32),
            pltpu.SemaphoreType.DMA,
            pltpu.SemaphoreType.DMA,
        ],
    )
    def k(y, ei, out, idxs, idxt, rowsA, rowsB, acc, semA, semB):
        c = lax.axis_index("c")
        s = lax.axis_index("s")
        wid = c * 16 + s
        _zero_vmem2d(rowsA, 128, HID)
        for b in range(5):
            pltpu.sync_copy(rowsA, acc.at[pl.ds(s * 640 + b * 128, 128)])
        plsc.subcore_barrier()

        def base(kk):
            return wid * EPT + kk * 128

        pltpu.sync_copy(ei.at[pl.ds(base(0), 128)], idxs.at[0])
        pltpu.sync_copy(ei.at[pl.ds(E + base(0), 128)], idxs.at[1])
        gA = pltpu.async_copy(y.at[idxs.at[0]], rowsA, semA)
        gB = None
        for kk in range(NFULL):
            last = kk == NFULL - 1
            if kk % 2 == 0:
                if not last:
                    pltpu.sync_copy(ei.at[pl.ds(base(kk + 1), 128)],
                                    idxs.at[2])
                    pltpu.sync_copy(ei.at[pl.ds(E + base(kk + 1), 128)],
                                    idxs.at[3])
                gA.wait()
                if not last:
                    gB = pltpu.async_copy(y.at[idxs.at[2]], rowsB, semB)
                pltpu.sync_copy(rowsA, acc.at[idxs.at[1]], add=True)
            else:
                if not last:
                    pltpu.sync_copy(ei.at[pl.ds(base(kk + 1), 128)],
                                    idxs.at[0])
                    pltpu.sync_copy(ei.at[pl.ds(E + base(kk + 1), 128)],
                                    idxs.at[1])
                gB.wait()
                if not last:
                    gA = pltpu.async_copy(y.at[idxs.at[0]], rowsA, semA)
                pltpu.sync_copy(rowsB, acc.at[idxs.at[3]], add=True)
        tb = wid * EPT + NFULL * 128
        pltpu.sync_copy(ei.at[pl.ds(tb, TAIL)], idxt.at[0])
        pltpu.sync_copy(ei.at[pl.ds(E + tb, TAIL)], idxt.at[1])
        pltpu.async_copy(y.at[idxt.at[0]], rowsA.at[pl.ds(0, TAIL)],
                         semA).wait()
        pltpu.sync_copy(rowsA.at[pl.ds(0, TAIL)], acc.at[idxt.at[1]],
                        add=True)
        plsc.subcore_barrier()
        for b in range(5):
            sl = pl.ds(s * 640 + b * 128, 128)
            pltpu.sync_copy(acc.at[sl], out.at[c, sl])

    return k


def make_agg_knn():
    CNT = M1PAD // 32
    CH = 120

    @functools.partial(
        pl.kernel,
        mesh=_mesh(),
        compiler_params=pltpu.CompilerParams(needs_layout_passes=False),
        out_type=jax.ShapeDtypeStruct((M1PAD, HID), jnp.float32),
        scratch_types=[
            pltpu.VMEM((6, CH), jnp.int32),
            pltpu.VMEM((CH, HID), jnp.float32),
            pltpu.VMEM((CH, HID), jnp.float32),
            pltpu.VMEM((CH, HID), jnp.float32),
            pltpu.SemaphoreType.DMA,
            pltpu.SemaphoreType.DMA,
        ],
    )
    def k(yy, nbrT, out, idxv, bufA, bufB, accv, semA, semB):
        c = lax.axis_index("c")
        s = lax.axis_index("s")
        wid = c * 16 + s
        bufs = (bufA, bufB)
        sems = (semA, semB)
        for t in range(CNT // CH):
            base = wid * CNT + t * CH
            for j in range(6):
                pltpu.sync_copy(nbrT.at[pl.ds(j * M1PAD + base, CH)],
                                idxv.at[j])
            g = [None] * 6
            g[0] = pltpu.async_copy(yy.at[idxv.at[0]], bufs[0], sems[0])
            for j in range(6):
                g[j].wait()
                if j < 5:
                    g[j + 1] = pltpu.async_copy(
                        yy.at[idxv.at[j + 1]], bufs[(j + 1) % 2],
                        sems[(j + 1) % 2])
                cur = bufs[j % 2]

                if j == 0:
                    def abody0(i, _):
                        r = i // 8
                        l = i % 8
                        sl = pl.ds(l * 16, 16)
                        accv[r, sl] = cur[r, sl]
                        return 0

                    lax.fori_loop(0, CH * 8, abody0, 0)
                else:
                    def abody(i, _):
                        r = i // 8
                        l = i % 8
                        sl = pl.ds(l * 16, 16)
                        accv[r, sl] = accv[r, sl] + cur[r, sl]
                        return 0

                    lax.fori_loop(0, CH * 8, abody, 0)
            pltpu.sync_copy(accv, out.at[pl.ds(base, CH)])

    return k


def make_topk(nvalid, K, bias, KOUT):
    NQ = NPAD // 16

    @functools.partial(
        pl.kernel,
        mesh=_mesh(),
        compiler_params=pltpu.CompilerParams(needs_layout_passes=False),
        out_type=[jax.ShapeDtypeStruct((KOUT,), jnp.int32),
                  jax.ShapeDtypeStruct((NPAD,), jnp.float32)],
        scratch_types=[
            pltpu.VMEM((2 * NPAD,), jnp.float32),
            pltpu.VMEM((NPAD,), jnp.float32),
            pltpu.VMEM((KOUT + 16,), jnp.int32),
            pltpu.VMEM((NPAD,), jnp.float32),
        ],
    )
    def k(histpair, out, outm, hpv, scorev, outv, maskv):
        c = lax.axis_index("c")
        s = lax.axis_index("s")

        @pl.when(jnp.logical_and(c == 0, s == 0))
        def _():
            pltpu.sync_copy(histpair, hpv)
            iota16 = lax.broadcasted_iota(jnp.int32, (16,), 0)

            def build(q, _):
                sl = pl.ds(q * 16, 16)
                gidx = q * 16 + iota16
                sv = (hpv[pl.ds(q * 16, 16)]
                      + hpv[pl.ds(NPAD + q * 16, 16)] + jnp.float32(bias))
                scorev[sl] = jnp.where(gidx < nvalid, sv, -1.0)
                return 0

            lax.fori_loop(0, NQ, build, 0)

            def count_ge(thr):
                def cbody(q, acc):
                    sv = scorev[pl.ds(q * 16, 16)]
                    return acc + jnp.where(sv >= thr, 1, 0)

                accv = lax.fori_loop(0, NQ, cbody, jnp.zeros((16,), jnp.int32))
                return jnp.sum(accv)

            def bbody(_, lohi):
                lo, hi = lohi
                mid = (lo + hi) // 2
                cnt = count_ge(mid.astype(jnp.float32))
                big = cnt >= K
                return (jnp.where(big, mid, lo), jnp.where(big, hi, mid))

            lo, hi = lax.fori_loop(
                0, 21, bbody, (jnp.int32(0), jnp.int32(1 << 20))
            )
            Tf = lo.astype(jnp.float32)
            n_gt = count_ge(Tf + 0.5)
            need_eq = K - n_gt

            def comp(q, carry):
                pos, eqseen = carry
                sl = pl.ds(q * 16, 16)
                sv = scorev[sl]
                gidx = q * 16 + iota16
                m_gt = sv >= Tf + 0.5
                m_eq = jnp.logical_and(sv >= Tf - 0.5, sv <= Tf + 0.5)
                eqc = plsc.cumsum(m_eq.astype(jnp.int32))
                keep = jnp.logical_or(
                    m_gt, jnp.logical_and(m_eq, (eqseen + eqc) <= need_eq)
                )
                maskv[sl] = jnp.where(keep, 1.0, 0.0)
                plsc.store_compressed(outv.at[pl.ds(pos, 16)], gidx, mask=keep)
                pos = pos + jnp.sum(keep.astype(jnp.int32))
                eqseen = eqseen + jnp.sum(m_eq.astype(jnp.int32))
                return (pos, eqseen)

            lax.fori_loop(0, NQ, comp, (jnp.int32(0), jnp.int32(0)))
            pltpu.sync_copy(outv.at[pl.ds(0, KOUT)], out)
            pltpu.sync_copy(maskv, outm)

    return k


def make_gather_rows(B, CH):
    CNT = B // 32

    @functools.partial(
        pl.kernel,
        mesh=_mesh(),
        compiler_params=pltpu.CompilerParams(needs_layout_passes=False),
        out_type=jax.ShapeDtypeStruct((B, HID), jnp.float32),
        scratch_types=[
            pltpu.VMEM((CH,), jnp.int32),
            pltpu.VMEM((CH, HID), jnp.float32),
            pltpu.SemaphoreType.DMA,
        ],
    )
    def k(table, idx, out, idxv, rowsv, sem):
        c = lax.axis_index("c")
        s = lax.axis_index("s")
        wid = c * 16 + s
        for t in range(CNT // CH):
            base = wid * CNT + t * CH
            pltpu.sync_copy(idx.at[pl.ds(base, CH)], idxv)
            pltpu.async_copy(table.at[idxv], rowsv, sem).wait()
            pltpu.sync_copy(rowsv, out.at[pl.ds(base, CH)])

    return k


def make_gather_cols():
    CNT = M1PAD // 32

    @functools.partial(
        pl.kernel,
        mesh=_mesh(),
        compiler_params=pltpu.CompilerParams(needs_layout_passes=False),
        out_type=jax.ShapeDtypeStruct((4 * M1PAD,), jnp.float32),
        scratch_types=[
            pltpu.VMEM((NPAD,), jnp.float32),
            pltpu.VMEM((CNT,), jnp.int32),
            pltpu.VMEM((CNT,), jnp.float32),
        ],
    )
    def k(colsT, idx, out, colv, idxv, outvv):
        c = lax.axis_index("c")
        s = lax.axis_index("s")
        wid = c * 16 + s
        base = wid * CNT
        iota16 = lax.broadcasted_iota(jnp.int32, (16,), 0)
        pltpu.sync_copy(idx.at[pl.ds(base, CNT)], idxv)
        for cc in range(4):
            pltpu.sync_copy(colsT.at[pl.ds(cc * NPAD, NPAD)], colv)
            fill = jnp.float32(PADPOS if cc < 3 else 0.0)

            def gbody(q, _):
                sl = pl.ds(q * 16, 16)
                iv = idxv[sl]
                vals = plsc.load_gather(colv, [iv])
                gidx = base + q * 16 + iota16
                outvv[sl] = jnp.where(gidx < M1, vals, fill)
                return 0

            lax.fori_loop(0, CNT // 16, gbody, 0)
            pltpu.sync_copy(outvv, out.at[pl.ds(cc * M1PAD + base, CNT)])

    return k


def make_scatter_rows(B, OUTR, CH, ZB):
    CNT = B // 16
    NCH = CNT // CH
    ZR = OUTR // 16

    @functools.partial(
        pl.kernel,
        mesh=_mesh(),
        compiler_params=pltpu.CompilerParams(needs_layout_passes=False),
        out_type=jax.ShapeDtypeStruct((OUTR, HID), jnp.float32),
        scratch_types=[
            pltpu.VMEM((NCH, CH), jnp.int32),
            pltpu.VMEM((CH, HID), jnp.float32),
            pltpu.VMEM((ZB, HID), jnp.float32),
        ],
    )
    def k(z, idx, out, idxv, rowsv, zbuf):
        c = lax.axis_index("c")
        s = lax.axis_index("s")

        @pl.when(c == 0)
        def _():
            _zero_vmem2d(zbuf, ZB, HID)
            for b in range(ZR // ZB):
                pltpu.sync_copy(zbuf, out.at[pl.ds(s * ZR + b * ZB, ZB)])
            plsc.subcore_barrier()
            for kk in range(NCH):
                base = s * CNT + kk * CH
                pltpu.sync_copy(idx.at[pl.ds(base, CH)], idxv.at[kk])
                pltpu.sync_copy(z.at[pl.ds(base, CH)], rowsv)
                pltpu.sync_copy(rowsv, out.at[idxv.at[kk]])

    return k


def make_mm(nrows, with_hist=False, with_scale=False, with_bias=False,
            relu=False):
    grid = nrows // 128

    def body(*refs):
        i = 0
        a_ref = refs[i]; i += 1
        w_ref = refs[i]; i += 1
        h_ref = None
        sc_ref = None
        b_ref = None
        if with_hist:
            h_ref = refs[i]; i += 1
        if with_scale:
            sc_ref = refs[i]; i += 1
        if with_bias:
            b_ref = refs[i]; i += 1
        out_ref = refs[i]; i += 1
        acc = jnp.dot(a_ref[...], w_ref[...],
                      preferred_element_type=jnp.float32)
        if with_hist:
            dinv = lax.rsqrt(h_ref[...] + 2.0)
            refs[i][...] = dinv
            acc = acc * dinv
        if with_scale:
            acc = acc * sc_ref[...]
        if with_bias:
            acc = acc + b_ref[...]
        if relu:
            acc = jnp.maximum(acc, 0.0)
        out_ref[...] = acc

    in_specs = [
        pl.BlockSpec((128, HID), lambda i: (i, 0)),
        pl.BlockSpec((HID, HID), lambda i: (0, 0)),
    ]
    if with_hist:
        in_specs.append(pl.BlockSpec((128, 1), lambda i: (i, 0)))
    if with_scale:
        in_specs.append(pl.BlockSpec((128, 1), lambda i: (i, 0)))
    if with_bias:
        in_specs.append(pl.BlockSpec((1, HID), lambda i: (0, 0)))
    out_shape = [jax.ShapeDtypeStruct((nrows, HID), jnp.float32)]
    out_specs = [pl.BlockSpec((128, HID), lambda i: (i, 0))]
    if with_hist:
        out_shape.append(jax.ShapeDtypeStruct((nrows, 1), jnp.float32))
        out_specs.append(pl.BlockSpec((128, 1), lambda i: (i, 0)))
    return pl.pallas_call(
        body,
        grid=(grid,),
        in_specs=in_specs,
        out_specs=out_specs if len(out_specs) > 1 else out_specs[0],
        out_shape=out_shape if len(out_shape) > 1 else out_shape[0],
    )


def make_post2(nrows):
    def body(sa, sb, y, dinv, b, out):
        d = dinv[...]
        out[...] = jnp.maximum(
            d * (sa[...] + sb[...]) + 2.0 * d * y[...] + b[...], 0.0
        )

    return pl.pallas_call(
        body,
        grid=(nrows // 128,),
        in_specs=[
            pl.BlockSpec((128, HID), lambda i: (i, 0)),
            pl.BlockSpec((128, HID), lambda i: (i, 0)),
            pl.BlockSpec((128, HID), lambda i: (i, 0)),
            pl.BlockSpec((128, 1), lambda i: (i, 0)),
            pl.BlockSpec((1, HID), lambda i: (0, 0)),
        ],
        out_specs=pl.BlockSpec((128, HID), lambda i: (i, 0)),
        out_shape=jax.ShapeDtypeStruct((nrows, HID), jnp.float32),
    )


def make_post1(nrows):
    def body(sref, y, b, out):
        out[...] = jnp.maximum(
            S8 * sref[...] + (2.0 * S8) * y[...] + b[...], 0.0
        )

    return pl.pallas_call(
        body,
        grid=(nrows // 128,),
        in_specs=[
            pl.BlockSpec((128, HID), lambda i: (i, 0)),
            pl.BlockSpec((128, HID), lambda i: (i, 0)),
            pl.BlockSpec((1, HID), lambda i: (0, 0)),
        ],
        out_specs=pl.BlockSpec((128, HID), lambda i: (i, 0)),
        out_shape=jax.ShapeDtypeStruct((nrows, HID), jnp.float32),
    )


def make_knn():
    BIGF = 3.0e38
    BIGI = 1 << 30

    def body(posR, posT, out):
        i = pl.program_id(0)
        pr = posR[...]
        a0 = pr[:, 0:1]
        a1 = pr[:, 1:2]
        a2 = pr[:, 2:3]
        c0 = posT[0:1, :]
        c1 = posT[1:2, :]
        c2 = posT[2:3, :]
        sq_r = a0 * a0 + a1 * a1 + a2 * a2
        sq_c = c0 * c0 + c1 * c1 + c2 * c2
        dots = lax.dot_general(pr, posT[...],
                               (((1,), (0,)), ((), ())))
        d = (sq_r + sq_c) - 2.0 * dots
        col = lax.broadcasted_iota(jnp.int32, (128, M1PAD), 1)
        row = i * 128 + lax.broadcasted_iota(jnp.int32, (128, M1PAD), 0)
        d = jnp.where(col == row, BIGF, d)
        for j in range(6):
            mn = jnp.min(d, axis=1, keepdims=True)
            amin = jnp.min(jnp.where(d == mn, col, BIGI), axis=1)
            out[j, :] = amin
            d = jnp.where(col == amin[:, None], BIGF, d)
        z = jnp.zeros((128,), jnp.int32)
        out[6, :] = z
        out[7, :] = z

    return pl.pallas_call(
        body,
        grid=(M1PAD // 128,),
        in_specs=[
            pl.BlockSpec((128, 3), lambda i: (i, 0)),
            pl.BlockSpec((3, M1PAD), lambda i: (0, 0)),
        ],
        out_specs=pl.BlockSpec((8, 128), lambda i: (0, i)),
        out_shape=jax.ShapeDtypeStruct((8, M1PAD), jnp.int32),
    )


@functools.lru_cache(maxsize=None)
def _ops():
    class O:
        pass
    o = O()
    o.hist_e0 = make_hist(2, E, 1, 2000, NPAD)
    o.hist_e1 = make_hist(8, M1PAD, 3, 480, NPAD)
    o.agg0 = make_agg_scatter()
    o.agg1 = make_agg_knn()
    o.topk1 = make_topk(N, M1, 0.0, K1PAD)
    o.topk2 = make_topk(M1, M2, 6.0, K2PAD)
    o.gather1 = make_gather_rows(M1PAD, 120)
    o.gather_cols = make_gather_cols()
    o.scatter_b = make_scatter_rows(M1PAD, NPAD, 120, 128)
    o.mm_y = make_mm(NPAD, with_hist=True)
    o.mm_1 = make_mm(M1PAD)
    o.mm_up1 = make_mm(M1PAD, with_scale=True)
    o.mm_out = make_mm(NPAD, with_bias=True)
    o.post2_0 = make_post2(NPAD)
    o.post1_1 = make_post1(M1PAD)
    o.knn = make_knn()
    return o


def kernel(x, pos, edge_index, batch, W_down0, b_down0, W_down1, b_down1,
           W_up0, b_up0, W_up1, b_up1, W_lin, b_lin):
    del batch
    o = _ops()
    ei = edge_index.astype(jnp.int32)
    xp = jnp.zeros((NPAD, HID), jnp.float32).at[:N].set(x)

    ei1d = ei.reshape(-1)
    histpair0 = o.hist_e0(ei1d)
    histd = histpair0[NPAD:].reshape(NPAD, 1)

    y, dinv0 = o.mm_y(xp, W_down0, histd)

    S = o.agg0(y, ei1d)
    h0 = o.post2_0(S[0], S[1], y, dinv0, b_down0.reshape(1, HID))

    perm1, _m1 = o.topk1(histpair0)
    perm1g = jnp.concatenate(
        [perm1[:M1], jnp.zeros((M1PAD - M1,), jnp.int32)])
    x1 = o.gather1(h0, perm1g)

    colsT = jnp.concatenate(
        [jnp.zeros((3, NPAD), jnp.float32).at[:, :N].set(pos.T),
         dinv0.reshape(1, NPAD)])
    cols1 = o.gather_cols(colsT.reshape(-1), perm1g).reshape(4, M1PAD)
    posT1 = cols1[:3]
    dinv0p = cols1[3].reshape(M1PAD, 1)

    nbrT = o.knn(posT1.T, posT1)

    nbr1d = nbrT.reshape(-1)
    y1 = o.mm_1(x1, W_down1 * S8)
    S1 = o.agg1(y1, nbr1d)
    h1 = o.post1_1(S1, y1, b_down1.reshape(1, HID))

    histpair1 = o.hist_e1(nbr1d)
    perm2, mask2 = o.topk2(histpair1)

    y_up = o.mm_up1(h1, W_up0 * S8, mask2[:M1PAD].reshape(M1PAD, 1))
    S2 = o.agg1(y_up, nbr1d)
    h_up0 = o.post1_1(S2, y_up, b_up0.reshape(1, HID))

    z2 = o.mm_up1(h_up0, W_up1, dinv0p)
    sidx1 = jnp.concatenate(
        [perm1[:M1], jnp.full((M1PAD - M1,), DUMP0, jnp.int32)])
    y2full = o.scatter_b(z2, sidx1)
    S3 = o.agg0(y2full, ei1d)
    h_up1 = o.post2_0(S3[0], S3[1], y2full, dinv0, b_up1.reshape(1, HID))

    out = o.mm_out(h_up1, W_lin, b_lin.reshape(1, HID))
    return out[:N]


# speedup vs baseline: 11.9868x; 1.0641x over previous
import functools

import jax
import jax.numpy as jnp
from jax import lax
from jax.experimental import pallas as pl
from jax.experimental.pallas import tpu as pltpu
from jax.experimental.pallas import tpu_sc as plsc

N = 10000
E = 320000
HID = 128
NPAD = 10240
M1, M1PAD = 7500, 7680
M2, M2PAD = 5625, 6144
K1PAD = 7504
K2PAD = 5632
DUMP1 = 7600
DUMP0 = 10100
PADPOS = 1.0e15
S8 = 0.35355339059327373

@functools.lru_cache(maxsize=None)
def _mesh():
    return plsc.VectorSubcoreMesh(
        core_axis_name="c", subcore_axis_name="s", num_cores=2,
        num_subcores=16)


def _zero_vmem2d(ref, rows, width):
    per = width // 16

    def body(i, _):
        r = i // per
        l = i % per
        ref[r, pl.ds(l * 16, 16)] = jnp.zeros((16,), ref.dtype)
        return 0

    lax.fori_loop(0, rows * per, body, 0)


def _zero_vmem1d(ref, n):
    def body(i, _):
        ref[pl.ds(i * 16, 16)] = jnp.zeros((16,), ref.dtype)
        return 0

    lax.fori_loop(0, n // 16, body, 0)


def make_hist(R, EC, rpc, CH, NB):
    PT = EC // 16
    NCH = PT // CH
    SL = NB // 16

    @functools.partial(
        pl.kernel,
        mesh=_mesh(),
        compiler_params=pltpu.CompilerParams(needs_layout_passes=False),
        out_type=jax.ShapeDtypeStruct((2 * NB,), jnp.float32),
        scratch_types=[
            pltpu.VMEM((NB,), jnp.float32),
            pltpu.VMEM((CH,), jnp.int32),
            pltpu.VMEM((SL,), jnp.float32),
            pltpu.VMEM((SL,), jnp.float32),
            pltpu.VMEM_SHARED((16, NB), jnp.float32),
        ],
    )
    def k(idx2, out, histv, idxv, accv, tmpv, stage):
        c = lax.axis_index("c")
        s = lax.axis_index("s")
        _zero_vmem1d(histv, NB)
        ones = jnp.ones((16,), jnp.float32)
        for r in range(rpc):
            row = c * rpc + r
            for kk in range(NCH):
                base = row * EC + s * PT + kk * CH
                pltpu.sync_copy(idx2.at[pl.ds(base, CH)], idxv)

                def sbody(j, _):
                    iv = idxv[pl.ds(j * 16, 16)]
                    plsc.addupdate_scatter(histv, [iv], ones)
                    return 0

                lax.fori_loop(0, CH // 16, sbody, 0)
        pltpu.sync_copy(histv, stage.at[s])
        plsc.subcore_barrier()
        off = s * SL
        pltpu.sync_copy(stage.at[0, pl.ds(off, SL)], accv)
        for t in range(1, 16):
            pltpu.sync_copy(stage.at[t, pl.ds(off, SL)], tmpv)

            def abody(q, _):
                sl = pl.ds(q * 16, 16)
                accv[sl] = accv[sl] + tmpv[sl]
                return 0

            lax.fori_loop(0, SL // 16, abody, 0)
        pltpu.sync_copy(accv, out.at[pl.ds(c * NB + off, SL)])

    return k


def make_agg_scatter():
    EPT = E // 32
    NFULL = EPT // 128
    TAIL = EPT - NFULL * 128

    @functools.partial(
        pl.kernel,
        mesh=_mesh(),
        compiler_params=pltpu.CompilerParams(needs_layout_passes=False),
        out_type=jax.ShapeDtypeStruct((2, NPAD, HID), jnp.float32),
        scratch_types=[
            pltpu.VMEM((4, 128), jnp.int32),
            pltpu.VMEM((2, 16), jnp.int32),
            pltpu.VMEM((128, HID), jnp.float32),
            pltpu.VMEM((128, HID), jnp.float32),
            pltpu.VMEM_SHARED((NPAD, HID), jnp.float32),
            pltpu.SemaphoreType.DMA,
            pltpu.SemaphoreType.DMA,
        ],
    )
    def k(y, ei, out, idxs, idxt, rowsA, rowsB, acc, semA, semB):
        c = lax.axis_index("c")
        s = lax.axis_index("s")
        wid = c * 16 + s
        _zero_vmem2d(rowsA, 128, HID)
        for b in range(5):
            pltpu.sync_copy(rowsA, acc.at[pl.ds(s * 640 + b * 128, 128)])
        plsc.subcore_barrier()

        def base(kk):
            return wid * EPT + kk * 128

        pltpu.sync_copy(ei.at[pl.ds(base(0), 128)], idxs.at[0])
        pltpu.sync_copy(ei.at[pl.ds(E + base(0), 128)], idxs.at[1])
        gA = pltpu.async_copy(y.at[idxs.at[0]], rowsA, semA)
        gB = None
        for kk in range(NFULL):
            last = kk == NFULL - 1
            if kk % 2 == 0:
                if not last:
                    pltpu.sync_copy(ei.at[pl.ds(base(kk + 1), 128)],
                                    idxs.at[2])
                    pltpu.sync_copy(ei.at[pl.ds(E + base(kk + 1), 128)],
                                    idxs.at[3])
                gA.wait()
                if not last:
                    gB = pltpu.async_copy(y.at[idxs.at[2]], rowsB, semB)
                pltpu.sync_copy(rowsA, acc.at[idxs.at[1]], add=True)
            else:
                if not last:
                    pltpu.sync_copy(ei.at[pl.ds(base(kk + 1), 128)],
                                    idxs.at[0])
                    pltpu.sync_copy(ei.at[pl.ds(E + base(kk + 1), 128)],
                                    idxs.at[1])
                gB.wait()
                if not last:
                    gA = pltpu.async_copy(y.at[idxs.at[0]], rowsA, semA)
                pltpu.sync_copy(rowsB, acc.at[idxs.at[3]], add=True)
        tb = wid * EPT + NFULL * 128
        pltpu.sync_copy(ei.at[pl.ds(tb, TAIL)], idxt.at[0])
        pltpu.sync_copy(ei.at[pl.ds(E + tb, TAIL)], idxt.at[1])
        pltpu.async_copy(y.at[idxt.at[0]], rowsA.at[pl.ds(0, TAIL)],
                         semA).wait()
        pltpu.sync_copy(rowsA.at[pl.ds(0, TAIL)], acc.at[idxt.at[1]],
                        add=True)
        plsc.subcore_barrier()
        for b in range(5):
            sl = pl.ds(s * 640 + b * 128, 128)
            pltpu.sync_copy(acc.at[sl], out.at[c, sl])

    return k


def make_agg_knn():
    CNT = M1PAD // 32
    CH = 120

    @functools.partial(
        pl.kernel,
        mesh=_mesh(),
        compiler_params=pltpu.CompilerParams(needs_layout_passes=False),
        out_type=jax.ShapeDtypeStruct((M1PAD, HID), jnp.float32),
        scratch_types=[
            pltpu.VMEM((6, CH), jnp.int32),
            pltpu.VMEM((CH, HID), jnp.float32),
            pltpu.VMEM((CH, HID), jnp.float32),
            pltpu.VMEM((CH, HID), jnp.float32),
            pltpu.SemaphoreType.DMA,
            pltpu.SemaphoreType.DMA,
        ],
    )
    def k(yy, nbrT, out, idxv, bufA, bufB, accv, semA, semB):
        c = lax.axis_index("c")
        s = lax.axis_index("s")
        wid = c * 16 + s
        bufs = (bufA, bufB)
        sems = (semA, semB)
        for t in range(CNT // CH):
            base = wid * CNT + t * CH
            for j in range(6):
                pltpu.sync_copy(nbrT.at[pl.ds(j * M1PAD + base, CH)],
                                idxv.at[j])
            g = [None] * 6
            g[0] = pltpu.async_copy(yy.at[idxv.at[0]], bufs[0], sems[0])
            for j in range(6):
                g[j].wait()
                if j < 5:
                    g[j + 1] = pltpu.async_copy(
                        yy.at[idxv.at[j + 1]], bufs[(j + 1) % 2],
                        sems[(j + 1) % 2])
                cur = bufs[j % 2]

                if j == 0:
                    def abody0(i, _):
                        r = i // 8
                        l = i % 8
                        sl = pl.ds(l * 16, 16)
                        accv[r, sl] = cur[r, sl]
                        return 0

                    lax.fori_loop(0, CH * 8, abody0, 0)
                else:
                    def abody(i, _):
                        r = i // 8
                        l = i % 8
                        sl = pl.ds(l * 16, 16)
                        accv[r, sl] = accv[r, sl] + cur[r, sl]
                        return 0

                    lax.fori_loop(0, CH * 8, abody, 0)
            pltpu.sync_copy(accv, out.at[pl.ds(base, CH)])

    return k


def make_topk(nvalid, K, bias, KOUT):
    NQ = NPAD // 16

    @functools.partial(
        pl.kernel,
        mesh=_mesh(),
        compiler_params=pltpu.CompilerParams(needs_layout_passes=False),
        out_type=[jax.ShapeDtypeStruct((KOUT,), jnp.int32),
                  jax.ShapeDtypeStruct((NPAD,), jnp.float32)],
        scratch_types=[
            pltpu.VMEM((2 * NPAD,), jnp.float32),
            pltpu.VMEM((NPAD,), jnp.float32),
            pltpu.VMEM((KOUT + 16,), jnp.int32),
            pltpu.VMEM((NPAD,), jnp.float32),
        ],
    )
    def k(histpair, out, outm, hpv, scorev, outv, maskv):
        c = lax.axis_index("c")
        s = lax.axis_index("s")

        @pl.when(jnp.logical_and(c == 0, s == 0))
        def _():
            pltpu.sync_copy(histpair, hpv)
            iota16 = lax.broadcasted_iota(jnp.int32, (16,), 0)

            def build(q, _):
                sl = pl.ds(q * 16, 16)
                gidx = q * 16 + iota16
                sv = (hpv[pl.ds(q * 16, 16)]
                      + hpv[pl.ds(NPAD + q * 16, 16)] + jnp.float32(bias))
                scorev[sl] = jnp.where(gidx < nvalid, sv, -1.0)
                return 0

            lax.fori_loop(0, NQ, build, 0)

            def count_ge(thr):
                def cbody(q, acc):
                    sv = scorev[pl.ds(q * 16, 16)]
                    return acc + jnp.where(sv >= thr, 1, 0)

                accv = lax.fori_loop(0, NQ, cbody, jnp.zeros((16,), jnp.int32))
                return jnp.sum(accv)

            def bbody(_, lohi):
                lo, hi = lohi
                mid = (lo + hi) // 2
                cnt = count_ge(mid.astype(jnp.float32))
                big = cnt >= K
                return (jnp.where(big, mid, lo), jnp.where(big, hi, mid))

            lo, hi = lax.fori_loop(
                0, 21, bbody, (jnp.int32(0), jnp.int32(1 << 20))
            )
            Tf = lo.astype(jnp.float32)
            n_gt = count_ge(Tf + 0.5)
            need_eq = K - n_gt

            def comp(q, carry):
                pos, eqseen = carry
                sl = pl.ds(q * 16, 16)
                sv = scorev[sl]
                gidx = q * 16 + iota16
                m_gt = sv >= Tf + 0.5
                m_eq = jnp.logical_and(sv >= Tf - 0.5, sv <= Tf + 0.5)
                eqc = plsc.cumsum(m_eq.astype(jnp.int32))
                keep = jnp.logical_or(
                    m_gt, jnp.logical_and(m_eq, (eqseen + eqc) <= need_eq)
                )
                maskv[sl] = jnp.where(keep, 1.0, 0.0)
                plsc.store_compressed(outv.at[pl.ds(pos, 16)], gidx, mask=keep)
                pos = pos + jnp.sum(keep.astype(jnp.int32))
                eqseen = eqseen + jnp.sum(m_eq.astype(jnp.int32))
                return (pos, eqseen)

            lax.fori_loop(0, NQ, comp, (jnp.int32(0), jnp.int32(0)))
            pltpu.sync_copy(outv.at[pl.ds(0, KOUT)], out)
            pltpu.sync_copy(maskv, outm)

    return k


def make_gather_rows(B, CH):
    CNT = B // 32

    @functools.partial(
        pl.kernel,
        mesh=_mesh(),
        compiler_params=pltpu.CompilerParams(needs_layout_passes=False),
        out_type=jax.ShapeDtypeStruct((B, HID), jnp.float32),
        scratch_types=[
            pltpu.VMEM((CH,), jnp.int32),
            pltpu.VMEM((CH, HID), jnp.float32),
            pltpu.SemaphoreType.DMA,
        ],
    )
    def k(table, idx, out, idxv, rowsv, sem):
        c = lax.axis_index("c")
        s = lax.axis_index("s")
        wid = c * 16 + s
        for t in range(CNT // CH):
            base = wid * CNT + t * CH
            pltpu.sync_copy(idx.at[pl.ds(base, CH)], idxv)
            pltpu.async_copy(table.at[idxv], rowsv, sem).wait()
            pltpu.sync_copy(rowsv, out.at[pl.ds(base, CH)])

    return k


def make_gather_cols():
    CNT = M1PAD // 32
    CH = 120

    @functools.partial(
        pl.kernel,
        mesh=_mesh(),
        compiler_params=pltpu.CompilerParams(needs_layout_passes=False),
        out_type=[jax.ShapeDtypeStruct((M1PAD, HID), jnp.float32),
                  jax.ShapeDtypeStruct((4 * M1PAD,), jnp.float32)],
        scratch_types=[
            pltpu.VMEM((NPAD,), jnp.float32),
            pltpu.VMEM((CNT,), jnp.int32),
            pltpu.VMEM((CNT,), jnp.float32),
            pltpu.VMEM((CH, HID), jnp.float32),
            pltpu.SemaphoreType.DMA,
        ],
    )
    def k(table, colsT, idx, outr, outc, colv, idxv, outvv, rowsv, sem):
        c = lax.axis_index("c")
        s = lax.axis_index("s")
        wid = c * 16 + s
        base = wid * CNT
        iota16 = lax.broadcasted_iota(jnp.int32, (16,), 0)
        pltpu.sync_copy(idx.at[pl.ds(base, CNT)], idxv)
        for t in range(CNT // CH):
            pltpu.async_copy(table.at[idxv.at[pl.ds(t * CH, CH)]], rowsv,
                             sem).wait()
            pltpu.sync_copy(rowsv, outr.at[pl.ds(base + t * CH, CH)])
        for cc in range(4):
            pltpu.sync_copy(colsT.at[pl.ds(cc * NPAD, NPAD)], colv)
            fill = jnp.float32(PADPOS if cc < 3 else 0.0)

            def gbody(q, _):
                sl = pl.ds(q * 16, 16)
                iv = idxv[sl]
                vals = plsc.load_gather(colv, [iv])
                gidx = base + q * 16 + iota16
                outvv[sl] = jnp.where(gidx < M1, vals, fill)
                return 0

            lax.fori_loop(0, CNT // 16, gbody, 0)
            pltpu.sync_copy(outvv, outc.at[pl.ds(cc * M1PAD + base, CNT)])

    return k


def make_scatter_rows(B, OUTR, CH, ZB):
    CNT = B // 16
    NCH = CNT // CH
    ZR = OUTR // 16

    @functools.partial(
        pl.kernel,
        mesh=_mesh(),
        compiler_params=pltpu.CompilerParams(needs_layout_passes=False),
        out_type=jax.ShapeDtypeStruct((OUTR, HID), jnp.float32),
        scratch_types=[
            pltpu.VMEM((NCH, CH), jnp.int32),
            pltpu.VMEM((CH, HID), jnp.float32),
            pltpu.VMEM((ZB, HID), jnp.float32),
        ],
    )
    def k(z, idx, out, idxv, rowsv, zbuf):
        c = lax.axis_index("c")
        s = lax.axis_index("s")

        @pl.when(c == 0)
        def _():
            _zero_vmem2d(zbuf, ZB, HID)
            for b in range(ZR // ZB):
                pltpu.sync_copy(zbuf, out.at[pl.ds(s * ZR + b * ZB, ZB)])
            plsc.subcore_barrier()
            for kk in range(NCH):
                base = s * CNT + kk * CH
                pltpu.sync_copy(idx.at[pl.ds(base, CH)], idxv.at[kk])
                pltpu.sync_copy(z.at[pl.ds(base, CH)], rowsv)
                pltpu.sync_copy(rowsv, out.at[idxv.at[kk]])

    return k


def make_mm(nrows, with_hist=False, with_scale=False, with_bias=False,
            relu=False):
    grid = nrows // 128

    def body(*refs):
        i = 0
        a_ref = refs[i]; i += 1
        w_ref = refs[i]; i += 1
        h_ref = None
        sc_ref = None
        b_ref = None
        if with_hist:
            h_ref = refs[i]; i += 1
        if with_scale:
            sc_ref = refs[i]; i += 1
        if with_bias:
            b_ref = refs[i]; i += 1
        out_ref = refs[i]; i += 1
        acc = jnp.dot(a_ref[...], w_ref[...],
                      preferred_element_type=jnp.float32)
        if with_hist:
            dinv = lax.rsqrt(h_ref[...] + 2.0)
            refs[i][...] = dinv
            acc = acc * dinv
        if with_scale:
            acc = acc * sc_ref[...]
        if with_bias:
            acc = acc + b_ref[...]
        if relu:
            acc = jnp.maximum(acc, 0.0)
        out_ref[...] = acc

    in_specs = [
        pl.BlockSpec((128, HID), lambda i: (i, 0)),
        pl.BlockSpec((HID, HID), lambda i: (0, 0)),
    ]
    if with_hist:
        in_specs.append(pl.BlockSpec((128, 1), lambda i: (i, 0)))
    if with_scale:
        in_specs.append(pl.BlockSpec((128, 1), lambda i: (i, 0)))
    if with_bias:
        in_specs.append(pl.BlockSpec((1, HID), lambda i: (0, 0)))
    out_shape = [jax.ShapeDtypeStruct((nrows, HID), jnp.float32)]
    out_specs = [pl.BlockSpec((128, HID), lambda i: (i, 0))]
    if with_hist:
        out_shape.append(jax.ShapeDtypeStruct((nrows, 1), jnp.float32))
        out_specs.append(pl.BlockSpec((128, 1), lambda i: (i, 0)))
    return pl.pallas_call(
        body,
        grid=(grid,),
        in_specs=in_specs,
        out_specs=out_specs if len(out_specs) > 1 else out_specs[0],
        out_shape=out_shape if len(out_shape) > 1 else out_shape[0],
    )


def make_post2(nrows):
    def body(sa, sb, y, dinv, b, out):
        d = dinv[...]
        out[...] = jnp.maximum(
            d * (sa[...] + sb[...]) + 2.0 * d * y[...] + b[...], 0.0
        )

    return pl.pallas_call(
        body,
        grid=(nrows // 128,),
        in_specs=[
            pl.BlockSpec((128, HID), lambda i: (i, 0)),
            pl.BlockSpec((128, HID), lambda i: (i, 0)),
            pl.BlockSpec((128, HID), lambda i: (i, 0)),
            pl.BlockSpec((128, 1), lambda i: (i, 0)),
            pl.BlockSpec((1, HID), lambda i: (0, 0)),
        ],
        out_specs=pl.BlockSpec((128, HID), lambda i: (i, 0)),
        out_shape=jax.ShapeDtypeStruct((nrows, HID), jnp.float32),
    )


def make_post1(nrows):
    def body(sref, y, b, out):
        out[...] = jnp.maximum(
            S8 * sref[...] + (2.0 * S8) * y[...] + b[...], 0.0
        )

    return pl.pallas_call(
        body,
        grid=(nrows // 128,),
        in_specs=[
            pl.BlockSpec((128, HID), lambda i: (i, 0)),
            pl.BlockSpec((128, HID), lambda i: (i, 0)),
            pl.BlockSpec((1, HID), lambda i: (0, 0)),
        ],
        out_specs=pl.BlockSpec((128, HID), lambda i: (i, 0)),
        out_shape=jax.ShapeDtypeStruct((nrows, HID), jnp.float32),
    )


def make_mm_post1(nrows):
    def body(sref, y, b, w, scale, out):
        h = jnp.maximum(S8 * sref[...] + (2.0 * S8) * y[...] + b[...], 0.0)
        out[...] = jnp.dot(h, w[...],
                           preferred_element_type=jnp.float32) * scale[...]

    return pl.pallas_call(
        body,
        grid=(nrows // 128,),
        in_specs=[
            pl.BlockSpec((128, HID), lambda i: (i, 0)),
            pl.BlockSpec((128, HID), lambda i: (i, 0)),
            pl.BlockSpec((1, HID), lambda i: (0, 0)),
            pl.BlockSpec((HID, HID), lambda i: (0, 0)),
            pl.BlockSpec((128, 1), lambda i: (i, 0)),
        ],
        out_specs=pl.BlockSpec((128, HID), lambda i: (i, 0)),
        out_shape=jax.ShapeDtypeStruct((nrows, HID), jnp.float32),
    )


def make_mm_post2_out(nrows):
    def body(sa, sb, y, dinv, b1, w, b2, out):
        d = dinv[...]
        h = jnp.maximum(
            d * (sa[...] + sb[...]) + 2.0 * d * y[...] + b1[...], 0.0)
        out[...] = jnp.dot(h, w[...],
                           preferred_element_type=jnp.float32) + b2[...]

    return pl.pallas_call(
        body,
        grid=(nrows // 128,),
        in_specs=[
            pl.BlockSpec((128, HID), lambda i: (i, 0)),
            pl.BlockSpec((128, HID), lambda i: (i, 0)),
            pl.BlockSpec((128, HID), lambda i: (i, 0)),
            pl.BlockSpec((128, 1), lambda i: (i, 0)),
            pl.BlockSpec((1, HID), lambda i: (0, 0)),
            pl.BlockSpec((HID, HID), lambda i: (0, 0)),
            pl.BlockSpec((1, HID), lambda i: (0, 0)),
        ],
        out_specs=pl.BlockSpec((128, HID), lambda i: (i, 0)),
        out_shape=jax.ShapeDtypeStruct((nrows, HID), jnp.float32),
    )


def make_knn():
    BIGF = 3.0e38
    BIGI = 1 << 30

    def body(posR, posT, out):
        i = pl.program_id(0)
        pr = posR[...]
        a0 = pr[:, 0:1]
        a1 = pr[:, 1:2]
        a2 = pr[:, 2:3]
        c0 = posT[0:1, :]
        c1 = posT[1:2, :]
        c2 = posT[2:3, :]
        sq_r = a0 * a0 + a1 * a1 + a2 * a2
        sq_c = c0 * c0 + c1 * c1 + c2 * c2
        dots = lax.dot_general(pr, posT[...],
                               (((1,), (0,)), ((), ())))
        d = (sq_r + sq_c) - 2.0 * dots
        col = lax.broadcasted_iota(jnp.int32, (128, M1PAD), 1)
        row = i * 128 + lax.broadcasted_iota(jnp.int32, (128, M1PAD), 0)
        d = jnp.where(col == row, BIGF, d)
        for j in range(6):
            mn = jnp.min(d, axis=1, keepdims=True)
            amin = jnp.min(jnp.where(d == mn, col, BIGI), axis=1)
            out[j, :] = amin
            d = jnp.where(col == amin[:, None], BIGF, d)
        z = jnp.zeros((128,), jnp.int32)
        out[6, :] = z
        out[7, :] = z

    return pl.pallas_call(
        body,
        grid=(M1PAD // 128,),
        in_specs=[
            pl.BlockSpec((128, 3), lambda i: (i, 0)),
            pl.BlockSpec((3, M1PAD), lambda i: (0, 0)),
        ],
        out_specs=pl.BlockSpec((8, 128), lambda i: (0, i)),
        out_shape=jax.ShapeDtypeStruct((8, M1PAD), jnp.int32),
    )


@functools.lru_cache(maxsize=None)
def _ops():
    class O:
        pass
    o = O()
    o.hist_e0 = make_hist(2, E, 1, 2000, NPAD)
    o.hist_e1 = make_hist(8, M1PAD, 3, 480, NPAD)
    o.agg0 = make_agg_scatter()
    o.agg1 = make_agg_knn()
    o.topk1 = make_topk(N, M1, 0.0, K1PAD)
    o.topk2 = make_topk(M1, M2, 6.0, K2PAD)
    o.gather_cols = make_gather_cols()
    o.scatter_b = make_scatter_rows(M1PAD, NPAD, 120, 128)
    o.mm_y = make_mm(NPAD, with_hist=True)
    o.mm_1 = make_mm(M1PAD)
    o.mm_post1 = make_mm_post1(M1PAD)
    o.mm_post2_out = make_mm_post2_out(NPAD)
    o.post2_0 = make_post2(NPAD)
    o.knn = make_knn()
    return o


def kernel(x, pos, edge_index, batch, W_down0, b_down0, W_down1, b_down1,
           W_up0, b_up0, W_up1, b_up1, W_lin, b_lin):
    del batch
    o = _ops()
    ei = edge_index.astype(jnp.int32)
    xp = jnp.zeros((NPAD, HID), jnp.float32).at[:N].set(x)

    ei1d = ei.reshape(-1)
    histpair0 = o.hist_e0(ei1d)
    histd = histpair0[NPAD:].reshape(NPAD, 1)

    y, dinv0 = o.mm_y(xp, W_down0, histd)

    S = o.agg0(y, ei1d)
    h0 = o.post2_0(S[0], S[1], y, dinv0, b_down0.reshape(1, HID))

    perm1, _m1 = o.topk1(histpair0)
    perm1g = jnp.concatenate(
        [perm1[:M1], jnp.zeros((M1PAD - M1,), jnp.int32)])
    colsT = jnp.concatenate(
        [jnp.zeros((3, NPAD), jnp.float32).at[:, :N].set(pos.T),
         dinv0.reshape(1, NPAD)])
    x1, cols1f = o.gather_cols(h0, colsT.reshape(-1), perm1g)
    cols1 = cols1f.reshape(4, M1PAD)
    posT1 = cols1[:3]
    dinv0p = cols1[3].reshape(M1PAD, 1)

    nbrT = o.knn(posT1.T, posT1)

    nbr1d = nbrT.reshape(-1)
    y1 = o.mm_1(x1, W_down1 * S8)
    S1 = o.agg1(y1, nbr1d)

    histpair1 = o.hist_e1(nbr1d)
    perm2, mask2 = o.topk2(histpair1)

    y_up = o.mm_post1(S1, y1, b_down1.reshape(1, HID), W_up0 * S8,
                      mask2[:M1PAD].reshape(M1PAD, 1))
    S2 = o.agg1(y_up, nbr1d)

    z2 = o.mm_post1(S2, y_up, b_up0.reshape(1, HID), W_up1, dinv0p)
    sidx1 = jnp.concatenate(
        [perm1[:M1], jnp.full((M1PAD - M1,), DUMP0, jnp.int32)])
    y2full = o.scatter_b(z2, sidx1)
    S3 = o.agg0(y2full, ei1d)
    out = o.mm_post2_out(S3[0], S3[1], y2full, dinv0,
                         b_up1.reshape(1, HID), W_lin,
                         b_lin.reshape(1, HID))
    return out[:N]


# speedup vs baseline: 12.5547x; 1.0474x over previous
import functools

import jax
import jax.numpy as jnp
from jax import lax
from jax.experimental import pallas as pl
from jax.experimental.pallas import tpu as pltpu
from jax.experimental.pallas import tpu_sc as plsc

N = 10000
E = 320000
HID = 128
NPAD = 10240
M1, M1PAD = 7500, 7680
M2, M2PAD = 5625, 6144
K1PAD = 7504
K2PAD = 5632
DUMP1 = 7600
DUMP0 = 10100
PADPOS = 1.0e15
S8 = 0.35355339059327373

@functools.lru_cache(maxsize=None)
def _mesh():
    return plsc.VectorSubcoreMesh(
        core_axis_name="c", subcore_axis_name="s", num_cores=2,
        num_subcores=16)


def _zero_vmem2d(ref, rows, width):
    per = width // 16

    def body(i, _):
        r = i // per
        l = i % per
        ref[r, pl.ds(l * 16, 16)] = jnp.zeros((16,), ref.dtype)
        return 0

    lax.fori_loop(0, rows * per, body, 0)


def _zero_vmem1d(ref, n):
    def body(i, _):
        ref[pl.ds(i * 16, 16)] = jnp.zeros((16,), ref.dtype)
        return 0

    lax.fori_loop(0, n // 16, body, 0)


def make_hist(R, EC, rpc, CH, NB):
    PT = EC // 16
    NCH = PT // CH
    SL = NB // 16

    @functools.partial(
        pl.kernel,
        mesh=_mesh(),
        compiler_params=pltpu.CompilerParams(needs_layout_passes=False),
        out_type=jax.ShapeDtypeStruct((2 * NB,), jnp.float32),
        scratch_types=[
            pltpu.VMEM((NB,), jnp.float32),
            pltpu.VMEM((CH,), jnp.int32),
            pltpu.VMEM((SL,), jnp.float32),
            pltpu.VMEM((16, SL), jnp.float32),
            pltpu.VMEM_SHARED((16, NB), jnp.float32),
        ],
    )
    def k(idx2, out, histv, idxv, accv, stgv, stage):
        c = lax.axis_index("c")
        s = lax.axis_index("s")
        _zero_vmem1d(histv, NB)
        ones = jnp.ones((16,), jnp.float32)
        for r in range(rpc):
            row = c * rpc + r
            for kk in range(NCH):
                base = row * EC + s * PT + kk * CH
                pltpu.sync_copy(idx2.at[pl.ds(base, CH)], idxv)

                def sbody(j, _):
                    iv = idxv[pl.ds(j * 16, 16)]
                    plsc.addupdate_scatter(histv, [iv], ones)
                    return 0

                lax.fori_loop(0, CH // 16, sbody, 0)
        pltpu.sync_copy(histv, stage.at[s])
        plsc.subcore_barrier()
        off = s * SL
        pltpu.sync_copy(stage.at[:, pl.ds(off, SL)], stgv)

        def abody(q, _):
            sl = pl.ds(q * 16, 16)
            acc = stgv[0, sl]
            for t in range(1, 16):
                acc = acc + stgv[t, sl]
            accv[sl] = acc
            return 0

        lax.fori_loop(0, SL // 16, abody, 0)
        pltpu.sync_copy(accv, out.at[pl.ds(c * NB + off, SL)])

    return k


def make_agg_scatter():
    EPT = E // 32
    NFULL = EPT // 128
    TAIL = EPT - NFULL * 128

    @functools.partial(
        pl.kernel,
        mesh=_mesh(),
        compiler_params=pltpu.CompilerParams(needs_layout_passes=False),
        out_type=jax.ShapeDtypeStruct((2, NPAD, HID), jnp.float32),
        scratch_types=[
            pltpu.VMEM((4, 128), jnp.int32),
            pltpu.VMEM((2, 16), jnp.int32),
            pltpu.VMEM((128, HID), jnp.float32),
            pltpu.VMEM((128, HID), jnp.float32),
            pltpu.VMEM_SHARED((NPAD, HID), jnp.float32),
            pltpu.SemaphoreType.DMA,
            pltpu.SemaphoreType.DMA,
        ],
    )
    def k(y, ei, out, idxs, idxt, rowsA, rowsB, acc, semA, semB):
        c = lax.axis_index("c")
        s = lax.axis_index("s")
        wid = c * 16 + s
        _zero_vmem2d(rowsA, 128, HID)
        for b in range(5):
            pltpu.sync_copy(rowsA, acc.at[pl.ds(s * 640 + b * 128, 128)])
        plsc.subcore_barrier()

        def base(kk):
            return wid * EPT + kk * 128

        pltpu.sync_copy(ei.at[pl.ds(base(0), 128)], idxs.at[0])
        pltpu.sync_copy(ei.at[pl.ds(E + base(0), 128)], idxs.at[1])
        gA = pltpu.async_copy(y.at[idxs.at[0]], rowsA, semA)
        gB = None
        for kk in range(NFULL):
            last = kk == NFULL - 1
            if kk % 2 == 0:
                if not last:
                    pltpu.sync_copy(ei.at[pl.ds(base(kk + 1), 128)],
                                    idxs.at[2])
                    pltpu.sync_copy(ei.at[pl.ds(E + base(kk + 1), 128)],
                                    idxs.at[3])
                gA.wait()
                if not last:
                    gB = pltpu.async_copy(y.at[idxs.at[2]], rowsB, semB)
                pltpu.sync_copy(rowsA, acc.at[idxs.at[1]], add=True)
            else:
                if not last:
                    pltpu.sync_copy(ei.at[pl.ds(base(kk + 1), 128)],
                                    idxs.at[0])
                    pltpu.sync_copy(ei.at[pl.ds(E + base(kk + 1), 128)],
                                    idxs.at[1])
                gB.wait()
                if not last:
                    gA = pltpu.async_copy(y.at[idxs.at[0]], rowsA, semA)
                pltpu.sync_copy(rowsB, acc.at[idxs.at[3]], add=True)
        tb = wid * EPT + NFULL * 128
        pltpu.sync_copy(ei.at[pl.ds(tb, TAIL)], idxt.at[0])
        pltpu.sync_copy(ei.at[pl.ds(E + tb, TAIL)], idxt.at[1])
        pltpu.async_copy(y.at[idxt.at[0]], rowsA.at[pl.ds(0, TAIL)],
                         semA).wait()
        pltpu.sync_copy(rowsA.at[pl.ds(0, TAIL)], acc.at[idxt.at[1]],
                        add=True)
        plsc.subcore_barrier()
        for b in range(5):
            sl = pl.ds(s * 640 + b * 128, 128)
            pltpu.sync_copy(acc.at[sl], out.at[c, sl])

    return k


def make_agg_knn():
    CNT = M1PAD // 32
    CH = 120

    @functools.partial(
        pl.kernel,
        mesh=_mesh(),
        compiler_params=pltpu.CompilerParams(needs_layout_passes=False),
        out_type=jax.ShapeDtypeStruct((M1PAD, HID), jnp.float32),
        scratch_types=[
            pltpu.VMEM((6, CH), jnp.int32),
            pltpu.VMEM((CH, HID), jnp.float32),
            pltpu.VMEM((CH, HID), jnp.float32),
            pltpu.VMEM((CH, HID), jnp.float32),
            pltpu.SemaphoreType.DMA,
            pltpu.SemaphoreType.DMA,
        ],
    )
    def k(yy, nbrT, out, idxv, bufA, bufB, accv, semA, semB):
        c = lax.axis_index("c")
        s = lax.axis_index("s")
        wid = c * 16 + s
        bufs = (bufA, bufB)
        sems = (semA, semB)
        for t in range(CNT // CH):
            base = wid * CNT + t * CH
            for j in range(6):
                pltpu.sync_copy(nbrT.at[pl.ds(j * M1PAD + base, CH)],
                                idxv.at[j])
            g = [None] * 6
            g[0] = pltpu.async_copy(yy.at[idxv.at[0]], bufs[0], sems[0])
            for j in range(6):
                g[j].wait()
                if j < 5:
                    g[j + 1] = pltpu.async_copy(
                        yy.at[idxv.at[j + 1]], bufs[(j + 1) % 2],
                        sems[(j + 1) % 2])
                cur = bufs[j % 2]

                if j == 0:
                    def abody0(i, _):
                        r = i // 8
                        l = i % 8
                        sl = pl.ds(l * 16, 16)
                        accv[r, sl] = cur[r, sl]
                        return 0

                    lax.fori_loop(0, CH * 8, abody0, 0)
                else:
                    def abody(i, _):
                        r = i // 8
                        l = i % 8
                        sl = pl.ds(l * 16, 16)
                        accv[r, sl] = accv[r, sl] + cur[r, sl]
                        return 0

                    lax.fori_loop(0, CH * 8, abody, 0)
            pltpu.sync_copy(accv, out.at[pl.ds(base, CH)])

    return k


def make_topk(nvalid, K, bias, KOUT):
    NQ = NPAD // 16

    @functools.partial(
        pl.kernel,
        mesh=_mesh(),
        compiler_params=pltpu.CompilerParams(needs_layout_passes=False),
        out_type=[jax.ShapeDtypeStruct((KOUT,), jnp.int32),
                  jax.ShapeDtypeStruct((NPAD,), jnp.float32)],
        scratch_types=[
            pltpu.VMEM((2 * NPAD,), jnp.float32),
            pltpu.VMEM((NPAD,), jnp.float32),
            pltpu.VMEM((KOUT + 16,), jnp.int32),
            pltpu.VMEM((NPAD,), jnp.float32),
        ],
    )
    def k(histpair, out, outm, hpv, scorev, outv, maskv):
        c = lax.axis_index("c")
        s = lax.axis_index("s")

        @pl.when(jnp.logical_and(c == 0, s == 0))
        def _():
            pltpu.sync_copy(histpair, hpv)
            iota16 = lax.broadcasted_iota(jnp.int32, (16,), 0)

            def build(q, _):
                sl = pl.ds(q * 16, 16)
                gidx = q * 16 + iota16
                sv = (hpv[pl.ds(q * 16, 16)]
                      + hpv[pl.ds(NPAD + q * 16, 16)] + jnp.float32(bias))
                scorev[sl] = jnp.where(gidx < nvalid, sv, -1.0)
                return 0

            lax.fori_loop(0, NQ, build, 0)

            def count_ge(thr):
                def cbody(q, acc):
                    sv = scorev[pl.ds(q * 16, 16)]
                    return acc + jnp.where(sv >= thr, 1, 0)

                accv = lax.fori_loop(0, NQ, cbody, jnp.zeros((16,), jnp.int32))
                return jnp.sum(accv)

            def mbody(q, acc):
                return jnp.maximum(acc, scorev[pl.ds(q * 16, 16)])

            maxv = lax.fori_loop(0, NQ, mbody,
                                 jnp.full((16,), -1.0, jnp.float32))
            hi0 = jnp.max(maxv).astype(jnp.int32) + 1

            def bcond(lohi):
                lo, hi = lohi
                return hi - lo > 1

            def bbody(lohi):
                lo, hi = lohi
                mid = (lo + hi) // 2
                cnt = count_ge(mid.astype(jnp.float32))
                big = cnt >= K
                return (jnp.where(big, mid, lo), jnp.where(big, hi, mid))

            lo, hi = lax.while_loop(bcond, bbody, (jnp.int32(0), hi0))
            Tf = lo.astype(jnp.float32)
            n_gt = count_ge(Tf + 0.5)
            need_eq = K - n_gt

            def comp(q, carry):
                pos, eqseen = carry
                sl = pl.ds(q * 16, 16)
                sv = scorev[sl]
                gidx = q * 16 + iota16
                m_gt = sv >= Tf + 0.5
                m_eq = jnp.logical_and(sv >= Tf - 0.5, sv <= Tf + 0.5)
                eqc = plsc.cumsum(m_eq.astype(jnp.int32))
                keep = jnp.logical_or(
                    m_gt, jnp.logical_and(m_eq, (eqseen + eqc) <= need_eq)
                )
                maskv[sl] = jnp.where(keep, 1.0, 0.0)
                plsc.store_compressed(outv.at[pl.ds(pos, 16)], gidx, mask=keep)
                pos = pos + jnp.sum(keep.astype(jnp.int32))
                eqseen = eqseen + jnp.sum(m_eq.astype(jnp.int32))
                return (pos, eqseen)

            lax.fori_loop(0, NQ, comp, (jnp.int32(0), jnp.int32(0)))
            pltpu.sync_copy(outv.at[pl.ds(0, KOUT)], out)
            pltpu.sync_copy(maskv, outm)

    return k


def make_gather_rows(B, CH):
    CNT = B // 32

    @functools.partial(
        pl.kernel,
        mesh=_mesh(),
        compiler_params=pltpu.CompilerParams(needs_layout_passes=False),
        out_type=jax.ShapeDtypeStruct((B, HID), jnp.float32),
        scratch_types=[
            pltpu.VMEM((CH,), jnp.int32),
            pltpu.VMEM((CH, HID), jnp.float32),
            pltpu.SemaphoreType.DMA,
        ],
    )
    def k(table, idx, out, idxv, rowsv, sem):
        c = lax.axis_index("c")
        s = lax.axis_index("s")
        wid = c * 16 + s
        for t in range(CNT // CH):
            base = wid * CNT + t * CH
            pltpu.sync_copy(idx.at[pl.ds(base, CH)], idxv)
            pltpu.async_copy(table.at[idxv], rowsv, sem).wait()
            pltpu.sync_copy(rowsv, out.at[pl.ds(base, CH)])

    return k


def make_gather_cols():
    CNT = M1PAD // 32
    CH = 120

    @functools.partial(
        pl.kernel,
        mesh=_mesh(),
        compiler_params=pltpu.CompilerParams(needs_layout_passes=False),
        out_type=[jax.ShapeDtypeStruct((M1PAD, HID), jnp.float32),
                  jax.ShapeDtypeStruct((4 * M1PAD,), jnp.float32)],
        scratch_types=[
            pltpu.VMEM((NPAD,), jnp.float32),
            pltpu.VMEM((CNT,), jnp.int32),
            pltpu.VMEM((CNT,), jnp.float32),
            pltpu.VMEM((CH, HID), jnp.float32),
            pltpu.SemaphoreType.DMA,
        ],
    )
    def k(table, colsT, idx, outr, outc, colv, idxv, outvv, rowsv, sem):
        c = lax.axis_index("c")
        s = lax.axis_index("s")
        wid = c * 16 + s
        base = wid * CNT
        iota16 = lax.broadcasted_iota(jnp.int32, (16,), 0)
        pltpu.sync_copy(idx.at[pl.ds(base, CNT)], idxv)
        for t in range(CNT // CH):
            pltpu.async_copy(table.at[idxv.at[pl.ds(t * CH, CH)]], rowsv,
                             sem).wait()
            pltpu.sync_copy(rowsv, outr.at[pl.ds(base + t * CH, CH)])
        for cc in range(4):
            pltpu.sync_copy(colsT.at[pl.ds(cc * NPAD, NPAD)], colv)
            fill = jnp.float32(PADPOS if cc < 3 else 0.0)

            def gbody(q, _):
                sl = pl.ds(q * 16, 16)
                iv = idxv[sl]
                vals = plsc.load_gather(colv, [iv])
                gidx = base + q * 16 + iota16
                outvv[sl] = jnp.where(gidx < M1, vals, fill)
                return 0

            lax.fori_loop(0, CNT // 16, gbody, 0)
            pltpu.sync_copy(outvv, outc.at[pl.ds(cc * M1PAD + base, CNT)])

    return k


def make_scatter_rows(B, OUTR, CH, ZB):
    CNT = B // 16
    NCH = CNT // CH
    ZR = OUTR // 16

    @functools.partial(
        pl.kernel,
        mesh=_mesh(),
        compiler_params=pltpu.CompilerParams(needs_layout_passes=False),
        out_type=jax.ShapeDtypeStruct((OUTR, HID), jnp.float32),
        scratch_types=[
            pltpu.VMEM((NCH, CH), jnp.int32),
            pltpu.VMEM((CH, HID), jnp.float32),
            pltpu.VMEM((ZB, HID), jnp.float32),
        ],
    )
    def k(z, idx, out, idxv, rowsv, zbuf):
        c = lax.axis_index("c")
        s = lax.axis_index("s")

        @pl.when(c == 0)
        def _():
            _zero_vmem2d(zbuf, ZB, HID)
            for b in range(ZR // ZB):
                pltpu.sync_copy(zbuf, out.at[pl.ds(s * ZR + b * ZB, ZB)])
            plsc.subcore_barrier()
            for kk in range(NCH):
                base = s * CNT + kk * CH
                pltpu.sync_copy(idx.at[pl.ds(base, CH)], idxv.at[kk])
                pltpu.sync_copy(z.at[pl.ds(base, CH)], rowsv)
                pltpu.sync_copy(rowsv, out.at[idxv.at[kk]])

    return k


def make_mm(nrows, with_hist=False, with_scale=False, with_bias=False,
            relu=False):
    grid = nrows // 128

    def body(*refs):
        i = 0
        a_ref = refs[i]; i += 1
        w_ref = refs[i]; i += 1
        h_ref = None
        sc_ref = None
        b_ref = None
        if with_hist:
            h_ref = refs[i]; i += 1
        if with_scale:
            sc_ref = refs[i]; i += 1
        if with_bias:
            b_ref = refs[i]; i += 1
        out_ref = refs[i]; i += 1
        acc = jnp.dot(a_ref[...], w_ref[...],
                      preferred_element_type=jnp.float32)
        if with_hist:
            dinv = lax.rsqrt(h_ref[...] + 2.0)
            refs[i][...] = dinv
            acc = acc * dinv
        if with_scale:
            acc = acc * sc_ref[...]
        if with_bias:
            acc = acc + b_ref[...]
        if relu:
            acc = jnp.maximum(acc, 0.0)
        out_ref[...] = acc

    in_specs = [
        pl.BlockSpec((128, HID), lambda i: (i, 0)),
        pl.BlockSpec((HID, HID), lambda i: (0, 0)),
    ]
    if with_hist:
        in_specs.append(pl.BlockSpec((128, 1), lambda i: (i, 0)))
    if with_scale:
        in_specs.append(pl.BlockSpec((128, 1), lambda i: (i, 0)))
    if with_bias:
        in_specs.append(pl.BlockSpec((1, HID), lambda i: (0, 0)))
    out_shape = [jax.ShapeDtypeStruct((nrows, HID), jnp.float32)]
    out_specs = [pl.BlockSpec((128, HID), lambda i: (i, 0))]
    if with_hist:
        out_shape.append(jax.ShapeDtypeStruct((nrows, 1), jnp.float32))
        out_specs.append(pl.BlockSpec((128, 1), lambda i: (i, 0)))
    return pl.pallas_call(
        body,
        grid=(grid,),
        in_specs=in_specs,
        out_specs=out_specs if len(out_specs) > 1 else out_specs[0],
        out_shape=out_shape if len(out_shape) > 1 else out_shape[0],
    )


def make_post2(nrows):
    def body(sa, sb, y, dinv, b, out):
        d = dinv[...]
        out[...] = jnp.maximum(
            d * (sa[...] + sb[...]) + 2.0 * d * y[...] + b[...], 0.0
        )

    return pl.pallas_call(
        body,
        grid=(nrows // 128,),
        in_specs=[
            pl.BlockSpec((128, HID), lambda i: (i, 0)),
            pl.BlockSpec((128, HID), lambda i: (i, 0)),
            pl.BlockSpec((128, HID), lambda i: (i, 0)),
            pl.BlockSpec((128, 1), lambda i: (i, 0)),
            pl.BlockSpec((1, HID), lambda i: (0, 0)),
        ],
        out_specs=pl.BlockSpec((128, HID), lambda i: (i, 0)),
        out_shape=jax.ShapeDtypeStruct((nrows, HID), jnp.float32),
    )


def make_post1(nrows):
    def body(sref, y, b, out):
        out[...] = jnp.maximum(
            S8 * sref[...] + (2.0 * S8) * y[...] + b[...], 0.0
        )

    return pl.pallas_call(
        body,
        grid=(nrows // 128,),
        in_specs=[
            pl.BlockSpec((128, HID), lambda i: (i, 0)),
            pl.BlockSpec((128, HID), lambda i: (i, 0)),
            pl.BlockSpec((1, HID), lambda i: (0, 0)),
        ],
        out_specs=pl.BlockSpec((128, HID), lambda i: (i, 0)),
        out_shape=jax.ShapeDtypeStruct((nrows, HID), jnp.float32),
    )


def make_mm_post1(nrows):
    def body(sref, y, b, w, scale, out):
        h = jnp.maximum(S8 * sref[...] + (2.0 * S8) * y[...] + b[...], 0.0)
        out[...] = jnp.dot(h, w[...],
                           preferred_element_type=jnp.float32) * scale[...]

    return pl.pallas_call(
        body,
        grid=(nrows // 128,),
        in_specs=[
            pl.BlockSpec((128, HID), lambda i: (i, 0)),
            pl.BlockSpec((128, HID), lambda i: (i, 0)),
            pl.BlockSpec((1, HID), lambda i: (0, 0)),
            pl.BlockSpec((HID, HID), lambda i: (0, 0)),
            pl.BlockSpec((128, 1), lambda i: (i, 0)),
        ],
        out_specs=pl.BlockSpec((128, HID), lambda i: (i, 0)),
        out_shape=jax.ShapeDtypeStruct((nrows, HID), jnp.float32),
    )


def make_mm_post2_out(nrows):
    def body(sa, sb, y, dinv, b1, w, b2, out):
        d = dinv[...]
        h = jnp.maximum(
            d * (sa[...] + sb[...]) + 2.0 * d * y[...] + b1[...], 0.0)
        out[...] = jnp.dot(h, w[...],
                           preferred_element_type=jnp.float32) + b2[...]

    return pl.pallas_call(
        body,
        grid=(nrows // 128,),
        in_specs=[
            pl.BlockSpec((128, HID), lambda i: (i, 0)),
            pl.BlockSpec((128, HID), lambda i: (i, 0)),
            pl.BlockSpec((128, HID), lambda i: (i, 0)),
            pl.BlockSpec((128, 1), lambda i: (i, 0)),
            pl.BlockSpec((1, HID), lambda i: (0, 0)),
            pl.BlockSpec((HID, HID), lambda i: (0, 0)),
            pl.BlockSpec((1, HID), lambda i: (0, 0)),
        ],
        out_specs=pl.BlockSpec((128, HID), lambda i: (i, 0)),
        out_shape=jax.ShapeDtypeStruct((nrows, HID), jnp.float32),
    )


def make_knn():
    BIGF = 3.0e38
    BIGI = 1 << 30

    def body(posR, posT, out):
        i = pl.program_id(0)
        pr = posR[...]
        a0 = pr[:, 0:1]
        a1 = pr[:, 1:2]
        a2 = pr[:, 2:3]
        c0 = posT[0:1, :]
        c1 = posT[1:2, :]
        c2 = posT[2:3, :]
        sq_r = a0 * a0 + a1 * a1 + a2 * a2
        sq_c = c0 * c0 + c1 * c1 + c2 * c2
        dots = lax.dot_general(pr, posT[...],
                               (((1,), (0,)), ((), ())))
        d = (sq_r + sq_c) - 2.0 * dots
        col = lax.broadcasted_iota(jnp.int32, (128, M1PAD), 1)
        row = i * 128 + lax.broadcasted_iota(jnp.int32, (128, M1PAD), 0)
        d = jnp.where(col == row, BIGF, d)
        for j in range(6):
            mn = jnp.min(d, axis=1, keepdims=True)
            amin = jnp.min(jnp.where(d == mn, col, BIGI), axis=1)
            out[j, :] = amin
            d = jnp.where(col == amin[:, None], BIGF, d)
        z = jnp.zeros((128,), jnp.int32)
        out[6, :] = z
        out[7, :] = z

    return pl.pallas_call(
        body,
        grid=(M1PAD // 128,),
        in_specs=[
            pl.BlockSpec((128, 3), lambda i: (i, 0)),
            pl.BlockSpec((3, M1PAD), lambda i: (0, 0)),
        ],
        out_specs=pl.BlockSpec((8, 128), lambda i: (0, i)),
        out_shape=jax.ShapeDtypeStruct((8, M1PAD), jnp.int32),
    )


@functools.lru_cache(maxsize=None)
def _ops():
    class O:
        pass
    o = O()
    o.hist_e0 = make_hist(2, E, 1, 2000, NPAD)
    o.hist_e1 = make_hist(8, M1PAD, 3, 480, NPAD)
    o.agg0 = make_agg_scatter()
    o.agg1 = make_agg_knn()
    o.topk1 = make_topk(N, M1, 0.0, K1PAD)
    o.topk2 = make_topk(M1, M2, 6.0, K2PAD)
    o.gather_cols = make_gather_cols()
    o.scatter_b = make_scatter_rows(M1PAD, NPAD, 120, 128)
    o.mm_y = make_mm(NPAD, with_hist=True)
    o.mm_1 = make_mm(M1PAD)
    o.mm_post1 = make_mm_post1(M1PAD)
    o.mm_post2_out = make_mm_post2_out(NPAD)
    o.post2_0 = make_post2(NPAD)
    o.knn = make_knn()
    return o


def kernel(x, pos, edge_index, batch, W_down0, b_down0, W_down1, b_down1,
           W_up0, b_up0, W_up1, b_up1, W_lin, b_lin):
    del batch
    o = _ops()
    ei = edge_index.astype(jnp.int32)
    xp = jnp.zeros((NPAD, HID), jnp.float32).at[:N].set(x)

    ei1d = ei.reshape(-1)
    histpair0 = o.hist_e0(ei1d)
    histd = histpair0[NPAD:].reshape(NPAD, 1)

    y, dinv0 = o.mm_y(xp, W_down0, histd)

    S = o.agg0(y, ei1d)
    h0 = o.post2_0(S[0], S[1], y, dinv0, b_down0.reshape(1, HID))

    perm1, _m1 = o.topk1(histpair0)
    perm1g = jnp.concatenate(
        [perm1[:M1], jnp.zeros((M1PAD - M1,), jnp.int32)])
    colsT = jnp.concatenate(
        [jnp.zeros((3, NPAD), jnp.float32).at[:, :N].set(pos.T),
         dinv0.reshape(1, NPAD)])
    x1, cols1f = o.gather_cols(h0, colsT.reshape(-1), perm1g)
    cols1 = cols1f.reshape(4, M1PAD)
    posT1 = cols1[:3]
    dinv0p = cols1[3].reshape(M1PAD, 1)

    nbrT = o.knn(posT1.T, posT1)

    nbr1d = nbrT.reshape(-1)
    y1 = o.mm_1(x1, W_down1 * S8)
    S1 = o.agg1(y1, nbr1d)

    histpair1 = o.hist_e1(nbr1d)
    perm2, mask2 = o.topk2(histpair1)

    y_up = o.mm_post1(S1, y1, b_down1.reshape(1, HID), W_up0 * S8,
                      mask2[:M1PAD].reshape(M1PAD, 1))
    S2 = o.agg1(y_up, nbr1d)

    z2 = o.mm_post1(S2, y_up, b_up0.reshape(1, HID), W_up1, dinv0p)
    sidx1 = jnp.concatenate(
        [perm1[:M1], jnp.full((M1PAD - M1,), DUMP0, jnp.int32)])
    y2full = o.scatter_b(z2, sidx1)
    S3 = o.agg0(y2full, ei1d)
    out = o.mm_post2_out(S3[0], S3[1], y2full, dinv0,
                         b_up1.reshape(1, HID), W_lin,
                         b_lin.reshape(1, HID))
    return out[:N]


# speedup vs baseline: 12.8030x; 1.0198x over previous
import functools

import jax
import jax.numpy as jnp
from jax import lax
from jax.experimental import pallas as pl
from jax.experimental.pallas import tpu as pltpu
from jax.experimental.pallas import tpu_sc as plsc

N = 10000
E = 320000
HID = 128
NPAD = 10240
M1, M1PAD = 7500, 7680
M2, M2PAD = 5625, 6144
K1PAD = 7504
K2PAD = 5632
DUMP1 = 7600
DUMP0 = 10100
PADPOS = 1.0e15
S8 = 0.35355339059327373

@functools.lru_cache(maxsize=None)
def _mesh():
    return plsc.VectorSubcoreMesh(
        core_axis_name="c", subcore_axis_name="s", num_cores=2,
        num_subcores=16)


def _zero_vmem2d(ref, rows, width):
    per = width // 16

    def body(r, _):
        for l in range(per):
            ref[r, pl.ds(l * 16, 16)] = jnp.zeros((16,), ref.dtype)
        return 0

    lax.fori_loop(0, rows, body, 0)


def _zero_vmem1d(ref, n):
    def body(i, _):
        ref[pl.ds(i * 16, 16)] = jnp.zeros((16,), ref.dtype)
        return 0

    lax.fori_loop(0, n // 16, body, 0)


def make_hist(R, EC, rpc, CH, NB):
    PT = EC // 16
    NCH = PT // CH
    SL = NB // 16

    @functools.partial(
        pl.kernel,
        mesh=_mesh(),
        compiler_params=pltpu.CompilerParams(needs_layout_passes=False),
        out_type=jax.ShapeDtypeStruct((2 * NB,), jnp.float32),
        scratch_types=[
            pltpu.VMEM((NB,), jnp.float32),
            pltpu.VMEM((CH,), jnp.int32),
            pltpu.VMEM((SL,), jnp.float32),
            pltpu.VMEM((16, SL), jnp.float32),
            pltpu.VMEM_SHARED((16, NB), jnp.float32),
        ],
    )
    def k(idx2, out, histv, idxv, accv, stgv, stage):
        c = lax.axis_index("c")
        s = lax.axis_index("s")
        _zero_vmem1d(histv, NB)
        ones = jnp.ones((16,), jnp.float32)
        for r in range(rpc):
            row = c * rpc + r
            for kk in range(NCH):
                base = row * EC + s * PT + kk * CH
                pltpu.sync_copy(idx2.at[pl.ds(base, CH)], idxv)

                def sbody(j, _):
                    iv = idxv[pl.ds(j * 16, 16)]
                    plsc.addupdate_scatter(histv, [iv], ones)
                    return 0

                lax.fori_loop(0, CH // 16, sbody, 0)
        pltpu.sync_copy(histv, stage.at[s])
        plsc.subcore_barrier()
        off = s * SL
        pltpu.sync_copy(stage.at[:, pl.ds(off, SL)], stgv)

        def abody(q, _):
            sl = pl.ds(q * 16, 16)
            acc = stgv[0, sl]
            for t in range(1, 16):
                acc = acc + stgv[t, sl]
            accv[sl] = acc
            return 0

        lax.fori_loop(0, SL // 16, abody, 0)
        pltpu.sync_copy(accv, out.at[pl.ds(c * NB + off, SL)])

    return k


def make_agg_scatter():
    EPT = E // 32
    NFULL = EPT // 128
    TAIL = EPT - NFULL * 128

    @functools.partial(
        pl.kernel,
        mesh=_mesh(),
        compiler_params=pltpu.CompilerParams(needs_layout_passes=False),
        out_type=jax.ShapeDtypeStruct((2, NPAD, HID), jnp.float32),
        scratch_types=[
            pltpu.VMEM((4, 128), jnp.int32),
            pltpu.VMEM((2, 16), jnp.int32),
            pltpu.VMEM((128, HID), jnp.float32),
            pltpu.VMEM((128, HID), jnp.float32),
            pltpu.VMEM_SHARED((NPAD, HID), jnp.float32),
            pltpu.SemaphoreType.DMA,
            pltpu.SemaphoreType.DMA,
        ],
    )
    def k(y, ei, out, idxs, idxt, rowsA, rowsB, acc, semA, semB):
        c = lax.axis_index("c")
        s = lax.axis_index("s")
        wid = c * 16 + s
        _zero_vmem2d(rowsA, 128, HID)
        for b in range(5):
            pltpu.sync_copy(rowsA, acc.at[pl.ds(s * 640 + b * 128, 128)])
        plsc.subcore_barrier()

        def base(kk):
            return wid * EPT + kk * 128

        pltpu.sync_copy(ei.at[pl.ds(base(0), 128)], idxs.at[0])
        pltpu.sync_copy(ei.at[pl.ds(E + base(0), 128)], idxs.at[1])
        gA = pltpu.async_copy(y.at[idxs.at[0]], rowsA, semA)
        gB = None
        for kk in range(NFULL):
            last = kk == NFULL - 1
            if kk % 2 == 0:
                if not last:
                    pltpu.sync_copy(ei.at[pl.ds(base(kk + 1), 128)],
                                    idxs.at[2])
                    pltpu.sync_copy(ei.at[pl.ds(E + base(kk + 1), 128)],
                                    idxs.at[3])
                gA.wait()
                if not last:
                    gB = pltpu.async_copy(y.at[idxs.at[2]], rowsB, semB)
                pltpu.sync_copy(rowsA, acc.at[idxs.at[1]], add=True)
            else:
                if not last:
                    pltpu.sync_copy(ei.at[pl.ds(base(kk + 1), 128)],
                                    idxs.at[0])
                    pltpu.sync_copy(ei.at[pl.ds(E + base(kk + 1), 128)],
                                    idxs.at[1])
                gB.wait()
                if not last:
                    gA = pltpu.async_copy(y.at[idxs.at[0]], rowsA, semA)
                pltpu.sync_copy(rowsB, acc.at[idxs.at[3]], add=True)
        tb = wid * EPT + NFULL * 128
        pltpu.sync_copy(ei.at[pl.ds(tb, TAIL)], idxt.at[0])
        pltpu.sync_copy(ei.at[pl.ds(E + tb, TAIL)], idxt.at[1])
        pltpu.async_copy(y.at[idxt.at[0]], rowsA.at[pl.ds(0, TAIL)],
                         semA).wait()
        pltpu.sync_copy(rowsA.at[pl.ds(0, TAIL)], acc.at[idxt.at[1]],
                        add=True)
        plsc.subcore_barrier()
        for b in range(5):
            sl = pl.ds(s * 640 + b * 128, 128)
            pltpu.sync_copy(acc.at[sl], out.at[c, sl])

    return k


def make_agg_knn():
    CNT = M1PAD // 32
    CH = 120

    @functools.partial(
        pl.kernel,
        mesh=_mesh(),
        compiler_params=pltpu.CompilerParams(needs_layout_passes=False),
        out_type=jax.ShapeDtypeStruct((M1PAD, HID), jnp.float32),
        scratch_types=[
            pltpu.VMEM((6, CH), jnp.int32),
            pltpu.VMEM((CH, HID), jnp.float32),
            pltpu.VMEM((CH, HID), jnp.float32),
            pltpu.VMEM((CH, HID), jnp.float32),
            pltpu.SemaphoreType.DMA,
            pltpu.SemaphoreType.DMA,
        ],
    )
    def k(yy, nbrT, out, idxv, bufA, bufB, accv, semA, semB):
        c = lax.axis_index("c")
        s = lax.axis_index("s")
        wid = c * 16 + s
        bufs = (bufA, bufB)
        sems = (semA, semB)
        for t in range(CNT // CH):
            base = wid * CNT + t * CH
            for j in range(6):
                pltpu.sync_copy(nbrT.at[pl.ds(j * M1PAD + base, CH)],
                                idxv.at[j])
            g = [None] * 6
            g[0] = pltpu.async_copy(yy.at[idxv.at[0]], bufs[0], sems[0])
            for j in range(6):
                g[j].wait()
                if j < 5:
                    g[j + 1] = pltpu.async_copy(
                        yy.at[idxv.at[j + 1]], bufs[(j + 1) % 2],
                        sems[(j + 1) % 2])
                cur = bufs[j % 2]

                if j == 0:
                    def abody0(r, _):
                        for l in range(8):
                            sl = pl.ds(l * 16, 16)
                            accv[r, sl] = cur[r, sl]
                        return 0

                    lax.fori_loop(0, CH, abody0, 0)
                else:
                    def abody(r, _):
                        for l in range(8):
                            sl = pl.ds(l * 16, 16)
                            accv[r, sl] = accv[r, sl] + cur[r, sl]
                        return 0

                    lax.fori_loop(0, CH, abody, 0)
            pltpu.sync_copy(accv, out.at[pl.ds(base, CH)])

    return k


def make_topk(nvalid, K, bias, KOUT):
    NQ = NPAD // 16

    @functools.partial(
        pl.kernel,
        mesh=_mesh(),
        compiler_params=pltpu.CompilerParams(needs_layout_passes=False),
        out_type=[jax.ShapeDtypeStruct((KOUT,), jnp.int32),
                  jax.ShapeDtypeStruct((NPAD,), jnp.float32)],
        scratch_types=[
            pltpu.VMEM((2 * NPAD,), jnp.float32),
            pltpu.VMEM((NPAD,), jnp.float32),
            pltpu.VMEM((KOUT + 16,), jnp.int32),
            pltpu.VMEM((NPAD,), jnp.float32),
        ],
    )
    def k(histpair, out, outm, hpv, scorev, outv, maskv):
        c = lax.axis_index("c")
        s = lax.axis_index("s")

        @pl.when(jnp.logical_and(c == 0, s == 0))
        def _():
            pltpu.sync_copy(histpair, hpv)
            iota16 = lax.broadcasted_iota(jnp.int32, (16,), 0)

            def build(q, _):
                sl = pl.ds(q * 16, 16)
                gidx = q * 16 + iota16
                sv = (hpv[pl.ds(q * 16, 16)]
                      + hpv[pl.ds(NPAD + q * 16, 16)] + jnp.float32(bias))
                scorev[sl] = jnp.where(gidx < nvalid, sv, -1.0)
                return 0

            lax.fori_loop(0, NQ, build, 0)

            def count_ge(thr):
                def cbody(q, acc):
                    sv = scorev[pl.ds(q * 16, 16)]
                    return acc + jnp.where(sv >= thr, 1, 0)

                accv = lax.fori_loop(0, NQ, cbody, jnp.zeros((16,), jnp.int32))
                return jnp.sum(accv)

            def mbody(q, acc):
                return jnp.maximum(acc, scorev[pl.ds(q * 16, 16)])

            maxv = lax.fori_loop(0, NQ, mbody,
                                 jnp.full((16,), -1.0, jnp.float32))
            hi0 = jnp.max(maxv).astype(jnp.int32) + 1

            def bcond(lohi):
                lo, hi = lohi
                return hi - lo > 1

            def bbody(lohi):
                lo, hi = lohi
                mid = (lo + hi) // 2
                cnt = count_ge(mid.astype(jnp.float32))
                big = cnt >= K
                return (jnp.where(big, mid, lo), jnp.where(big, hi, mid))

            lo, hi = lax.while_loop(bcond, bbody, (jnp.int32(0), hi0))
            Tf = lo.astype(jnp.float32)
            n_gt = count_ge(Tf + 0.5)
            need_eq = K - n_gt

            def comp(q, carry):
                pos, eqseen = carry
                sl = pl.ds(q * 16, 16)
                sv = scorev[sl]
                gidx = q * 16 + iota16
                m_gt = sv >= Tf + 0.5
                m_eq = jnp.logical_and(sv >= Tf - 0.5, sv <= Tf + 0.5)
                eqc = plsc.cumsum(m_eq.astype(jnp.int32))
                keep = jnp.logical_or(
                    m_gt, jnp.logical_and(m_eq, (eqseen + eqc) <= need_eq)
                )
                maskv[sl] = jnp.where(keep, 1.0, 0.0)
                plsc.store_compressed(outv.at[pl.ds(pos, 16)], gidx, mask=keep)
                pos = pos + jnp.sum(keep.astype(jnp.int32))
                eqseen = eqseen + jnp.sum(m_eq.astype(jnp.int32))
                return (pos, eqseen)

            lax.fori_loop(0, NQ, comp, (jnp.int32(0), jnp.int32(0)))
            pltpu.sync_copy(outv.at[pl.ds(0, KOUT)], out)
            pltpu.sync_copy(maskv, outm)

    return k


def make_gather_rows(B, CH):
    CNT = B // 32

    @functools.partial(
        pl.kernel,
        mesh=_mesh(),
        compiler_params=pltpu.CompilerParams(needs_layout_passes=False),
        out_type=jax.ShapeDtypeStruct((B, HID), jnp.float32),
        scratch_types=[
            pltpu.VMEM((CH,), jnp.int32),
            pltpu.VMEM((CH, HID), jnp.float32),
            pltpu.SemaphoreType.DMA,
        ],
    )
    def k(table, idx, out, idxv, rowsv, sem):
        c = lax.axis_index("c")
        s = lax.axis_index("s")
        wid = c * 16 + s
        for t in range(CNT // CH):
            base = wid * CNT + t * CH
            pltpu.sync_copy(idx.at[pl.ds(base, CH)], idxv)
            pltpu.async_copy(table.at[idxv], rowsv, sem).wait()
            pltpu.sync_copy(rowsv, out.at[pl.ds(base, CH)])

    return k


def make_gather_cols():
    CNT = M1PAD // 32
    CH = 120

    @functools.partial(
        pl.kernel,
        mesh=_mesh(),
        compiler_params=pltpu.CompilerParams(needs_layout_passes=False),
        out_type=[jax.ShapeDtypeStruct((M1PAD, HID), jnp.float32),
                  jax.ShapeDtypeStruct((4 * M1PAD,), jnp.float32)],
        scratch_types=[
            pltpu.VMEM((NPAD,), jnp.float32),
            pltpu.VMEM((CNT,), jnp.int32),
            pltpu.VMEM((CNT,), jnp.float32),
            pltpu.VMEM((CH, HID), jnp.float32),
            pltpu.SemaphoreType.DMA,
        ],
    )
    def k(table, colsT, idx, outr, outc, colv, idxv, outvv, rowsv, sem):
        c = lax.axis_index("c")
        s = lax.axis_index("s")
        wid = c * 16 + s
        base = wid * CNT
        iota16 = lax.broadcasted_iota(jnp.int32, (16,), 0)
        pltpu.sync_copy(idx.at[pl.ds(base, CNT)], idxv)
        for t in range(CNT // CH):
            pltpu.async_copy(table.at[idxv.at[pl.ds(t * CH, CH)]], rowsv,
                             sem).wait()
            pltpu.sync_copy(rowsv, outr.at[pl.ds(base + t * CH, CH)])
        for cc in range(4):
            pltpu.sync_copy(colsT.at[pl.ds(cc * NPAD, NPAD)], colv)
            fill = jnp.float32(PADPOS if cc < 3 else 0.0)

            def gbody(q, _):
                sl = pl.ds(q * 16, 16)
                iv = idxv[sl]
                vals = plsc.load_gather(colv, [iv])
                gidx = base + q * 16 + iota16
                outvv[sl] = jnp.where(gidx < M1, vals, fill)
                return 0

            lax.fori_loop(0, CNT // 16, gbody, 0)
            pltpu.sync_copy(outvv, outc.at[pl.ds(cc * M1PAD + base, CNT)])

    return k


def make_scatter_rows(B, OUTR, CH, ZB):
    CNT = B // 16
    NCH = CNT // CH
    ZR = OUTR // 16

    @functools.partial(
        pl.kernel,
        mesh=_mesh(),
        compiler_params=pltpu.CompilerParams(needs_layout_passes=False),
        out_type=jax.ShapeDtypeStruct((OUTR, HID), jnp.float32),
        scratch_types=[
            pltpu.VMEM((NCH, CH), jnp.int32),
            pltpu.VMEM((CH, HID), jnp.float32),
            pltpu.VMEM((ZB, HID), jnp.float32),
        ],
    )
    def k(z, idx, out, idxv, rowsv, zbuf):
        c = lax.axis_index("c")
        s = lax.axis_index("s")

        @pl.when(c == 0)
        def _():
            _zero_vmem2d(zbuf, ZB, HID)
            for b in range(ZR // ZB):
                pltpu.sync_copy(zbuf, out.at[pl.ds(s * ZR + b * ZB, ZB)])
            plsc.subcore_barrier()
            for kk in range(NCH):
                base = s * CNT + kk * CH
                pltpu.sync_copy(idx.at[pl.ds(base, CH)], idxv.at[kk])
                pltpu.sync_copy(z.at[pl.ds(base, CH)], rowsv)
                pltpu.sync_copy(rowsv, out.at[idxv.at[kk]])

    return k


def make_mm(nrows, with_hist=False, with_scale=False, with_bias=False,
            relu=False):
    grid = nrows // 128

    def body(*refs):
        i = 0
        a_ref = refs[i]; i += 1
        w_ref = refs[i]; i += 1
        h_ref = None
        sc_ref = None
        b_ref = None
        if with_hist:
            h_ref = refs[i]; i += 1
        if with_scale:
            sc_ref = refs[i]; i += 1
        if with_bias:
            b_ref = refs[i]; i += 1
        out_ref = refs[i]; i += 1
        acc = jnp.dot(a_ref[...], w_ref[...],
                      preferred_element_type=jnp.float32)
        if with_hist:
            dinv = lax.rsqrt(h_ref[...] + 2.0)
            refs[i][...] = dinv
            acc = acc * dinv
        if with_scale:
            acc = acc * sc_ref[...]
        if with_bias:
            acc = acc + b_ref[...]
        if relu:
            acc = jnp.maximum(acc, 0.0)
        out_ref[...] = acc

    in_specs = [
        pl.BlockSpec((128, HID), lambda i: (i, 0)),
        pl.BlockSpec((HID, HID), lambda i: (0, 0)),
    ]
    if with_hist:
        in_specs.append(pl.BlockSpec((128, 1), lambda i: (i, 0)))
    if with_scale:
        in_specs.append(pl.BlockSpec((128, 1), lambda i: (i, 0)))
    if with_bias:
        in_specs.append(pl.BlockSpec((1, HID), lambda i: (0, 0)))
    out_shape = [jax.ShapeDtypeStruct((nrows, HID), jnp.float32)]
    out_specs = [pl.BlockSpec((128, HID), lambda i: (i, 0))]
    if with_hist:
        out_shape.append(jax.ShapeDtypeStruct((nrows, 1), jnp.float32))
        out_specs.append(pl.BlockSpec((128, 1), lambda i: (i, 0)))
    return pl.pallas_call(
        body,
        grid=(grid,),
        in_specs=in_specs,
        out_specs=out_specs if len(out_specs) > 1 else out_specs[0],
        out_shape=out_shape if len(out_shape) > 1 else out_shape[0],
    )


def make_post2(nrows):
    def body(sa, sb, y, dinv, b, out):
        d = dinv[...]
        out[...] = jnp.maximum(
            d * (sa[...] + sb[...]) + 2.0 * d * y[...] + b[...], 0.0
        )

    return pl.pallas_call(
        body,
        grid=(nrows // 128,),
        in_specs=[
            pl.BlockSpec((128, HID), lambda i: (i, 0)),
            pl.BlockSpec((128, HID), lambda i: (i, 0)),
            pl.BlockSpec((128, HID), lambda i: (i, 0)),
            pl.BlockSpec((128, 1), lambda i: (i, 0)),
            pl.BlockSpec((1, HID), lambda i: (0, 0)),
        ],
        out_specs=pl.BlockSpec((128, HID), lambda i: (i, 0)),
        out_shape=jax.ShapeDtypeStruct((nrows, HID), jnp.float32),
    )


def make_post1(nrows):
    def body(sref, y, b, out):
        out[...] = jnp.maximum(
            S8 * sref[...] + (2.0 * S8) * y[...] + b[...], 0.0
        )

    return pl.pallas_call(
        body,
        grid=(nrows // 128,),
        in_specs=[
            pl.BlockSpec((128, HID), lambda i: (i, 0)),
            pl.BlockSpec((128, HID), lambda i: (i, 0)),
            pl.BlockSpec((1, HID), lambda i: (0, 0)),
        ],
        out_specs=pl.BlockSpec((128, HID), lambda i: (i, 0)),
        out_shape=jax.ShapeDtypeStruct((nrows, HID), jnp.float32),
    )


def make_mm_post1(nrows):
    def body(sref, y, b, w, scale, out):
        h = jnp.maximum(S8 * sref[...] + (2.0 * S8) * y[...] + b[...], 0.0)
        out[...] = jnp.dot(h, w[...],
                           preferred_element_type=jnp.float32) * scale[...]

    return pl.pallas_call(
        body,
        grid=(nrows // 128,),
        in_specs=[
            pl.BlockSpec((128, HID), lambda i: (i, 0)),
            pl.BlockSpec((128, HID), lambda i: (i, 0)),
            pl.BlockSpec((1, HID), lambda i: (0, 0)),
            pl.BlockSpec((HID, HID), lambda i: (0, 0)),
            pl.BlockSpec((128, 1), lambda i: (i, 0)),
        ],
        out_specs=pl.BlockSpec((128, HID), lambda i: (i, 0)),
        out_shape=jax.ShapeDtypeStruct((nrows, HID), jnp.float32),
    )


def make_mm_post2_out(nrows):
    def body(sa, sb, y, dinv, b1, w, b2, out):
        d = dinv[...]
        h = jnp.maximum(
            d * (sa[...] + sb[...]) + 2.0 * d * y[...] + b1[...], 0.0)
        out[...] = jnp.dot(h, w[...],
                           preferred_element_type=jnp.float32) + b2[...]

    return pl.pallas_call(
        body,
        grid=(nrows // 128,),
        in_specs=[
            pl.BlockSpec((128, HID), lambda i: (i, 0)),
            pl.BlockSpec((128, HID), lambda i: (i, 0)),
            pl.BlockSpec((128, HID), lambda i: (i, 0)),
            pl.BlockSpec((128, 1), lambda i: (i, 0)),
            pl.BlockSpec((1, HID), lambda i: (0, 0)),
            pl.BlockSpec((HID, HID), lambda i: (0, 0)),
            pl.BlockSpec((1, HID), lambda i: (0, 0)),
        ],
        out_specs=pl.BlockSpec((128, HID), lambda i: (i, 0)),
        out_shape=jax.ShapeDtypeStruct((nrows, HID), jnp.float32),
    )


def make_knn():
    BIGF = 3.0e38
    BIGI = 1 << 30

    def body(posR, posT, out):
        i = pl.program_id(0)
        pr = posR[...]
        a0 = pr[:, 0:1]
        a1 = pr[:, 1:2]
        a2 = pr[:, 2:3]
        c0 = posT[0:1, :]
        c1 = posT[1:2, :]
        c2 = posT[2:3, :]
        sq_r = a0 * a0 + a1 * a1 + a2 * a2
        sq_c = c0 * c0 + c1 * c1 + c2 * c2
        dots = lax.dot_general(pr, posT[...],
                               (((1,), (0,)), ((), ())))
        d = (sq_r + sq_c) - 2.0 * dots
        col = lax.broadcasted_iota(jnp.int32, (128, M1PAD), 1)
        row = i * 128 + lax.broadcasted_iota(jnp.int32, (128, M1PAD), 0)
        d = jnp.where(col == row, BIGF, d)
        for j in range(6):
            mn = jnp.min(d, axis=1, keepdims=True)
            amin = jnp.min(jnp.where(d == mn, col, BIGI), axis=1)
            out[j, :] = amin
            d = jnp.where(col == amin[:, None], BIGF, d)
        z = jnp.zeros((128,), jnp.int32)
        out[6, :] = z
        out[7, :] = z

    return pl.pallas_call(
        body,
        grid=(M1PAD // 128,),
        in_specs=[
            pl.BlockSpec((128, 3), lambda i: (i, 0)),
            pl.BlockSpec((3, M1PAD), lambda i: (0, 0)),
        ],
        out_specs=pl.BlockSpec((8, 128), lambda i: (0, i)),
        out_shape=jax.ShapeDtypeStruct((8, M1PAD), jnp.int32),
    )


@functools.lru_cache(maxsize=None)
def _ops():
    class O:
        pass
    o = O()
    o.hist_e0 = make_hist(2, E, 1, 2000, NPAD)
    o.hist_e1 = make_hist(8, M1PAD, 3, 480, NPAD)
    o.agg0 = make_agg_scatter()
    o.agg1 = make_agg_knn()
    o.topk1 = make_topk(N, M1, 0.0, K1PAD)
    o.topk2 = make_topk(M1, M2, 6.0, K2PAD)
    o.gather_cols = make_gather_cols()
    o.scatter_b = make_scatter_rows(M1PAD, NPAD, 120, 128)
    o.mm_y = make_mm(NPAD, with_hist=True)
    o.mm_1 = make_mm(M1PAD)
    o.mm_post1 = make_mm_post1(M1PAD)
    o.mm_post2_out = make_mm_post2_out(NPAD)
    o.post2_0 = make_post2(NPAD)
    o.knn = make_knn()
    return o


def kernel(x, pos, edge_index, batch, W_down0, b_down0, W_down1, b_down1,
           W_up0, b_up0, W_up1, b_up1, W_lin, b_lin):
    del batch
    o = _ops()
    ei = edge_index.astype(jnp.int32)
    xp = jnp.zeros((NPAD, HID), jnp.float32).at[:N].set(x)

    ei1d = ei.reshape(-1)
    histpair0 = o.hist_e0(ei1d)
    histd = histpair0[NPAD:].reshape(NPAD, 1)

    y, dinv0 = o.mm_y(xp, W_down0, histd)

    S = o.agg0(y, ei1d)
    h0 = o.post2_0(S[0], S[1], y, dinv0, b_down0.reshape(1, HID))

    perm1, _m1 = o.topk1(histpair0)
    perm1g = jnp.concatenate(
        [perm1[:M1], jnp.zeros((M1PAD - M1,), jnp.int32)])
    colsT = jnp.concatenate(
        [jnp.zeros((3, NPAD), jnp.float32).at[:, :N].set(pos.T),
         dinv0.reshape(1, NPAD)])
    x1, cols1f = o.gather_cols(h0, colsT.reshape(-1), perm1g)
    cols1 = cols1f.reshape(4, M1PAD)
    posT1 = cols1[:3]
    dinv0p = cols1[3].reshape(M1PAD, 1)

    nbrT = o.knn(posT1.T, posT1)

    nbr1d = nbrT.reshape(-1)
    y1 = o.mm_1(x1, W_down1 * S8)
    S1 = o.agg1(y1, nbr1d)

    histpair1 = o.hist_e1(nbr1d)
    perm2, mask2 = o.topk2(histpair1)

    y_up = o.mm_post1(S1, y1, b_down1.reshape(1, HID), W_up0 * S8,
                      mask2[:M1PAD].reshape(M1PAD, 1))
    S2 = o.agg1(y_up, nbr1d)

    z2 = o.mm_post1(S2, y_up, b_up0.reshape(1, HID), W_up1, dinv0p)
    sidx1 = jnp.concatenate(
        [perm1[:M1], jnp.full((M1PAD - M1,), DUMP0, jnp.int32)])
    y2full = o.scatter_b(z2, sidx1)
    S3 = o.agg0(y2full, ei1d)
    out = o.mm_post2_out(S3[0], S3[1], y2full, dinv0,
                         b_up1.reshape(1, HID), W_lin,
                         b_lin.reshape(1, HID))
    return out[:N]
